# Initial kernel scaffold; baseline (speedup 1.0000x reference)
import jax
import jax.numpy as jnp
from jax.experimental import pallas as pl


def kernel(x, edge_index, W1, b1, p1, W2, b2, p2):
    raise NotImplementedError("write your pallas kernel here")


# speedup vs baseline: 1.0284x; 1.0284x over previous
import functools

import jax
import jax.numpy as jnp
import numpy as np
from jax.experimental import pallas as pl
from jax.experimental.pallas import tpu as pltpu

N_NODES = 10000
N_EDGES = 320000
D_FEAT = 128
H1 = 126
H2 = 62
RATIO = 0.5


def _mm_body(x_ref, w_ref, o_ref):
    o_ref[...] = jnp.dot(x_ref[...], w_ref[...], preferred_element_type=jnp.float32)


def _matmul(x, w, block_m):
    m, k = x.shape
    _, n = w.shape
    return pl.pallas_call(
        _mm_body,
        out_shape=jax.ShapeDtypeStruct((m, n), jnp.float32),
        grid=(m // block_m,),
        in_specs=[
            pl.BlockSpec((block_m, k), lambda i: (i, 0)),
            pl.BlockSpec((k, n), lambda i: (0, 0)),
        ],
        out_specs=pl.BlockSpec((block_m, n), lambda i: (i, 0)),
    )(x, w)


def _gcn_conv(x, src, dst, ew, W, b):
    N = x.shape[0]
    xw = _matmul(x, W, 2000 if N == N_NODES else 1000)
    loop = jnp.arange(N, dtype=src.dtype)
    s = jnp.concatenate([src, loop])
    d = jnp.concatenate([dst, loop])
    w = jnp.concatenate([ew, jnp.ones((N,), x.dtype)])
    deg = jnp.zeros((N,), x.dtype).at[d].add(w)
    dinv = jnp.where(deg > 0, jax.lax.rsqrt(jnp.where(deg > 0, deg, 1.0)), 0.0)
    norm = dinv[s] * w * dinv[d]
    out = jnp.zeros_like(xw).at[d].add(norm[:, None] * xw[s])
    return out + b


def _topk_pool(x, src, dst, ew, p, ratio):
    N = x.shape[0]
    k = int(np.ceil(ratio * N))
    score = jnp.tanh((x @ p) / jnp.linalg.norm(p))
    vals, perm = jax.lax.top_k(score, k)
    x_new = x[perm] * vals[:, None]
    mask = jnp.zeros((N,), bool).at[perm].set(True)
    new_id = jnp.zeros((N,), jnp.int32).at[perm].set(jnp.arange(k, dtype=jnp.int32))
    valid = mask[src] & mask[dst]
    new_src = jnp.where(valid, new_id[src], 0)
    new_dst = jnp.where(valid, new_id[dst], 0)
    new_ew = ew * valid.astype(x.dtype)
    return x_new, new_src, new_dst, new_ew


def kernel(x, edge_index, W1, b1, p1, W2, b2, p2):
    src, dst = edge_index[0], edge_index[1]
    ew = jnp.ones((src.shape[0],), x.dtype)
    h = jax.nn.relu(_gcn_conv(x, src, dst, ew, W1, b1))
    h, src, dst, ew = _topk_pool(h, src, dst, ew, p1, RATIO)
    h = jax.nn.relu(_gcn_conv(h, src, dst, ew, W2, b2))
    h, src, dst, ew = _topk_pool(h, src, dst, ew, p2, RATIO)
    return jax.nn.log_softmax(h, axis=1)


# speedup vs baseline: 2.5925x; 2.5208x over previous
import functools

import jax
import jax.numpy as jnp
import numpy as np
from jax import lax
from jax.experimental import pallas as pl
from jax.experimental.pallas import tpu as pltpu
from jax.experimental.pallas import tpu_sc as plsc

N = 10000
NPAD = 10240
E = 320000
NW = 32
WN = 125
WL = 80
EPT = WN * WL
K1 = 5000
K2 = 2500


def _mesh():
    return plsc.VectorSubcoreMesh(core_axis_name="c", subcore_axis_name="s")


@functools.partial(
    pl.kernel,
    out_type=jax.ShapeDtypeStruct((2, NPAD), jnp.float32),
    mesh=_mesh(),
    scratch_types=[
        pltpu.VMEM((WN, WL), jnp.int32),
        pltpu.VMEM((WN, WL), jnp.float32),
        pltpu.VMEM((NPAD // 16,), jnp.float32),
        pltpu.VMEM_SHARED((NPAD,), jnp.float32),
    ],
)
def _sc_hist(idx_hbm, val_hbm, out_hbm, idx_v, val_v, z_v, acc_sh):
    c = lax.axis_index("c")
    s = lax.axis_index("s")
    wid = s * 2 + c
    sl = NPAD // 16

    def zbody(i, carry):
        z_v[pl.ds(i * 16, 16)] = jnp.zeros((16,), jnp.float32)
        return carry

    lax.fori_loop(0, sl // 16, zbody, 0)
    pltpu.sync_copy(z_v, acc_sh.at[pl.ds(s * sl, sl)])
    plsc.subcore_barrier()

    pltpu.sync_copy(idx_hbm.at[wid], idx_v)
    pltpu.sync_copy(val_hbm.at[wid], val_v)

    def body(w, carry):
        pltpu.sync_copy(val_v.at[w], acc_sh.at[idx_v.at[w]], add=True)
        return carry

    lax.fori_loop(0, WN, body, 0)
    plsc.subcore_barrier()
    pltpu.sync_copy(acc_sh.at[pl.ds(s * sl, sl)], out_hbm.at[c, pl.ds(s * sl, sl)])


def _make_msgpass(d_feat):
    @functools.partial(
        pl.kernel,
        out_type=jax.ShapeDtypeStruct((2, NPAD, d_feat), jnp.float32),
        mesh=_mesh(),
        compiler_params=pltpu.CompilerParams(use_tc_tiling_on_sc=False),
        scratch_types=[
            pltpu.VMEM((WN, WL), jnp.int32),
            pltpu.VMEM((WN, WL), jnp.int32),
            pltpu.VMEM((WL, d_feat), jnp.float32),
            pltpu.VMEM((WL, d_feat), jnp.float32),
            pltpu.VMEM_SHARED((NPAD, d_feat), jnp.float32),
            pltpu.SemaphoreType.DMA,
        ],
    )
    def msgpass(y_hbm, src_hbm, dst_hbm, out_hbm, sidx_v, didx_v, zb_v, rbuf_v,
                acc_sh, sem):
        c = lax.axis_index("c")
        s = lax.axis_index("s")
        wid = s * 2 + c
        rows = NPAD // 16

        def zbody(i, carry):
            for u in range(d_feat // 16):
                zb_v[i, pl.ds(u * 16, 16)] = jnp.zeros((16,), jnp.float32)
            return carry

        lax.fori_loop(0, WL, zbody, 0)
        for t in range(rows // WL):
            pltpu.sync_copy(zb_v, acc_sh.at[pl.ds(s * rows + t * WL, WL)])
        plsc.subcore_barrier()

        pltpu.sync_copy(src_hbm.at[wid], sidx_v)
        pltpu.sync_copy(dst_hbm.at[wid], didx_v)

        def body(w, carry):
            pltpu.async_copy(y_hbm.at[sidx_v.at[w]], rbuf_v, sem).wait()
            pltpu.sync_copy(rbuf_v, acc_sh.at[didx_v.at[w]], add=True)
            return carry

        lax.fori_loop(0, WN, body, 0)
        plsc.subcore_barrier()
        pltpu.sync_copy(acc_sh.at[pl.ds(s * rows, rows)],
                        out_hbm.at[c, pl.ds(s * rows, rows)])

    return msgpass


_msgpass64 = _make_msgpass(64)


def _mm_body(x_ref, w_ref, o_ref):
    o_ref[...] = jnp.dot(x_ref[...], w_ref[...], preferred_element_type=jnp.float32)


def _matmul(x, w, block_m):
    m, k = x.shape
    _, n = w.shape
    return pl.pallas_call(
        _mm_body,
        out_shape=jax.ShapeDtypeStruct((m, n), jnp.float32),
        grid=(m // block_m,),
        in_specs=[
            pl.BlockSpec((block_m, k), lambda i: (i, 0)),
            pl.BlockSpec((k, n), lambda i: (0, 0)),
        ],
        out_specs=pl.BlockSpec((block_m, n), lambda i: (i, 0)),
    )(x, w)


def kernel(x, edge_index, W1, b1, p1, W2, b2, p2):
    f32 = jnp.float32
    src, dst = edge_index[0], edge_index[1]
    src_r = src.reshape(NW, WN, WL)
    dst_r = dst.reshape(NW, WN, WL)
    ones_r = jnp.ones((NW, WN, WL), f32)

    deg1 = _sc_hist(dst_r, ones_r).sum(0)[:N] + 1.0
    dinv1 = lax.rsqrt(deg1)
    dinv1p = jnp.concatenate([dinv1, jnp.zeros((NPAD - N,), f32)])
    xpad = jnp.pad(x, ((0, NPAD - N), (0, 0)))
    W1p = jnp.pad(W1, ((0, 0), (0, 128 - W1.shape[1])))
    xw1 = _matmul(xpad, W1p, 1280)
    y1 = xw1 * dinv1p[:, None]
    acc_lo = _msgpass64(y1[:, :64], src_r, dst_r)
    acc_hi = _msgpass64(y1[:, 64:], src_r, dst_r)
    acc = jnp.concatenate([acc_lo[0] + acc_lo[1], acc_hi[0] + acc_hi[1]], axis=1)
    h = jax.nn.relu((dinv1p[:, None] * (acc + y1))[:N, :126] + b1)

    score = jnp.tanh((h @ p1) / jnp.linalg.norm(p1))
    vals, perm = lax.top_k(score, K1)
    h1 = h[perm] * vals[:, None]
    mask = jnp.zeros((N,), bool).at[perm].set(True)
    new_id = jnp.zeros((N,), jnp.int32).at[perm].set(jnp.arange(K1, dtype=jnp.int32))
    valid = mask[src] & mask[dst]

    eidx = jnp.arange(E, dtype=jnp.int32)
    gs = jnp.where(valid, new_id[src], K1 + (eidx % 5120))
    gd = jnp.where(valid, new_id[dst], 5120 + (eidx % 5120))
    gs_r = gs.reshape(NW, WN, WL)
    gd_r = gd.reshape(NW, WN, WL)
    deg2 = _sc_hist(gd_r, valid.astype(f32).reshape(NW, WN, WL)).sum(0)[:K1] + 1.0
    dinv2 = lax.rsqrt(deg2)
    W2p = jnp.pad(W2, ((0, 2), (0, 64 - W2.shape[1])))
    h1p = jnp.pad(h1, ((0, 120), (0, 2)))
    xw2 = _matmul(h1p, W2p, 1280)[:K1]
    z2 = xw2 * dinv2[:, None]
    z2ext = jnp.pad(z2, ((0, NPAD - K1), (0, 0)))
    acc2p = _msgpass64(z2ext, gs_r, gd_r)
    h2 = jax.nn.relu((dinv2[:, None] * ((acc2p[0] + acc2p[1])[:K1] + z2))[:, :62] + b2)

    score2 = jnp.tanh((h2 @ p2) / jnp.linalg.norm(p2))
    vals2, perm2 = lax.top_k(score2, K2)
    h2p = h2[perm2] * vals2[:, None]
    return jax.nn.log_softmax(h2p, axis=1)


# speedup vs baseline: 31.0813x; 11.9891x over previous
import functools

import jax
import jax.numpy as jnp
import numpy as np
from jax import lax
from jax.experimental import pallas as pl
from jax.experimental.pallas import tpu as pltpu
from jax.experimental.pallas import tpu_sc as plsc

N = 10000
NPAD = 10240
E = 320000
NW = 32
WN = 125
WL = 80
K1 = 5000
K2 = 2500
NPOOL2 = 5120


def _mesh():
    return plsc.VectorSubcoreMesh(core_axis_name="c", subcore_axis_name="s")


_SC_PARAMS = pltpu.CompilerParams(use_tc_tiling_on_sc=False)
_SC_PARAMS_NL = pltpu.CompilerParams(use_tc_tiling_on_sc=False,
                                     needs_layout_passes=False)


@functools.partial(
    pl.kernel,
    out_type=jax.ShapeDtypeStruct((2, NPAD), jnp.float32),
    mesh=_mesh(),
    compiler_params=_SC_PARAMS,
    scratch_types=[
        pltpu.VMEM((WN, WL), jnp.int32),
        pltpu.VMEM((WL,), jnp.float32),
        pltpu.VMEM((NPAD // 16,), jnp.float32),
        pltpu.VMEM_SHARED((NPAD,), jnp.float32),
    ],
)
def _sc_hist(idx_hbm, out_hbm, idx_v, ones_v, z_v, acc_sh):
    c = lax.axis_index("c")
    s = lax.axis_index("s")
    wid = s * 2 + c
    sl = NPAD // 16

    def zbody(i, carry):
        z_v[pl.ds(i * 16, 16)] = jnp.zeros((16,), jnp.float32)
        return carry

    lax.fori_loop(0, sl // 16, zbody, 0)
    for u in range(WL // 16):
        ones_v[pl.ds(u * 16, 16)] = jnp.ones((16,), jnp.float32)
    pltpu.sync_copy(z_v, acc_sh.at[pl.ds(s * sl, sl)])
    plsc.subcore_barrier()

    pltpu.sync_copy(idx_hbm.at[wid], idx_v)

    def body(w, carry):
        pltpu.sync_copy(ones_v, acc_sh.at[idx_v.at[w]], add=True)
        return carry

    lax.fori_loop(0, WN, body, 0)
    plsc.subcore_barrier()
    pltpu.sync_copy(acc_sh.at[pl.ds(s * sl, sl)], out_hbm.at[c, pl.ds(s * sl, sl)])


def _make_msgpass(d_feat):
    @functools.partial(
        pl.kernel,
        out_type=jax.ShapeDtypeStruct((2, NPAD, d_feat), jnp.float32),
        mesh=_mesh(),
        compiler_params=_SC_PARAMS,
        scratch_types=[
            pltpu.VMEM((WN, WL), jnp.int32),
            pltpu.VMEM((WN, WL), jnp.int32),
            pltpu.VMEM((WL, d_feat), jnp.float32),
            pltpu.VMEM((WL, d_feat), jnp.float32),
            pltpu.VMEM_SHARED((NPAD, d_feat), jnp.float32),
            pltpu.SemaphoreType.DMA,
        ],
    )
    def msgpass(y_hbm, src_hbm, dst_hbm, out_hbm, sidx_v, didx_v, zb_v, rbuf_v,
                acc_sh, sem):
        c = lax.axis_index("c")
        s = lax.axis_index("s")
        wid = s * 2 + c
        rows = NPAD // 16

        def zbody(i, carry):
            for u in range(d_feat // 16):
                zb_v[i, pl.ds(u * 16, 16)] = jnp.zeros((16,), jnp.float32)
            return carry

        lax.fori_loop(0, WL, zbody, 0)
        for t in range(rows // WL):
            pltpu.sync_copy(zb_v, acc_sh.at[pl.ds(s * rows + t * WL, WL)])
        plsc.subcore_barrier()

        pltpu.sync_copy(src_hbm.at[wid], sidx_v)
        pltpu.sync_copy(dst_hbm.at[wid], didx_v)

        def body(w, carry):
            pltpu.async_copy(y_hbm.at[sidx_v.at[w]], rbuf_v, sem).wait()
            pltpu.sync_copy(rbuf_v, acc_sh.at[didx_v.at[w]], add=True)
            return carry

        lax.fori_loop(0, WN, body, 0)
        plsc.subcore_barrier()
        pltpu.sync_copy(acc_sh.at[pl.ds(s * rows, rows)],
                        out_hbm.at[c, pl.ds(s * rows, rows)])

    return msgpass


_msgpass64 = _make_msgpass(64)


@functools.partial(
    pl.kernel,
    out_type=(
        jax.ShapeDtypeStruct((NPAD, 128), jnp.float32),
        jax.ShapeDtypeStruct((NW, WN, WL), jnp.int32),
        jax.ShapeDtypeStruct((NW, WN, WL), jnp.int32),
        jax.ShapeDtypeStruct((2, NPAD), jnp.float32),
    ),
    mesh=_mesh(),
    compiler_params=_SC_PARAMS_NL,
    scratch_types=[
        pltpu.VMEM((NPAD,), jnp.int32),
        pltpu.VMEM((64,), jnp.int32),
        pltpu.VMEM((64, 128), jnp.float32),
        pltpu.VMEM((WN, WL), jnp.int32),
        pltpu.VMEM((WN, WL), jnp.int32),
        pltpu.VMEM((WN, WL), jnp.int32),
        pltpu.VMEM((WN, WL), jnp.int32),
        pltpu.VMEM((WN, WL), jnp.float32),
        pltpu.VMEM((NPAD // 16,), jnp.float32),
        pltpu.VMEM_SHARED((NPAD,), jnp.float32),
        pltpu.SemaphoreType.DMA,
    ],
)
def _sc_pool1(hs_hbm, rank_hbm, src_hbm, dst_hbm,
              h1_hbm, gs_hbm, gd_hbm, deg2_hbm,
              rank_v, rkw_v, rowbuf_v, sidx_v, didx_v, gsv, gdv, ksv, z_v,
              deg_sh, sem):
    c = lax.axis_index("c")
    s = lax.axis_index("s")
    wid = s * 2 + c
    sl = NPAD // 16

    def zbody(i, carry):
        z_v[pl.ds(i * 16, 16)] = jnp.zeros((16,), jnp.float32)
        return carry

    lax.fori_loop(0, sl // 16, zbody, 0)
    pltpu.sync_copy(z_v, deg_sh.at[pl.ds(s * sl, sl)])
    plsc.subcore_barrier()

    pltpu.sync_copy(rank_hbm, rank_v)
    for w in range(5):
        for u in range(4):
            rkw_v[pl.ds(u * 16, 16)] = rank_v[pl.ds(wid * 320 + w * 64 + u * 16, 16)]
        pltpu.async_copy(hs_hbm.at[pl.ds(wid * 320 + w * 64, 64)], rowbuf_v,
                         sem).wait()
        pltpu.sync_copy(rowbuf_v, h1_hbm.at[rkw_v])

    pltpu.sync_copy(src_hbm.at[wid], sidx_v)
    pltpu.sync_copy(dst_hbm.at[wid], didx_v)

    def body(w, carry):
        for u in range(WL // 16):
            s16 = sidx_v[w, pl.ds(u * 16, 16)]
            d16 = didx_v[w, pl.ds(u * 16, 16)]
            gs16 = plsc.load_gather(rank_v, [s16])
            gd16 = plsc.load_gather(rank_v, [d16])
            ks16 = jnp.where(gs16 < K1, 1.0, 0.0).astype(jnp.float32)
            gsv[w, pl.ds(u * 16, 16)] = gs16
            gdv[w, pl.ds(u * 16, 16)] = gd16
            ksv[w, pl.ds(u * 16, 16)] = ks16
        pltpu.sync_copy(ksv.at[w], deg_sh.at[gdv.at[w]], add=True)
        return carry

    lax.fori_loop(0, WN, body, 0)
    pltpu.sync_copy(gsv, gs_hbm.at[wid])
    pltpu.sync_copy(gdv, gd_hbm.at[wid])
    plsc.subcore_barrier()
    pltpu.sync_copy(deg_sh.at[pl.ds(s * sl, sl)], deg2_hbm.at[c, pl.ds(s * sl, sl)])


@functools.partial(
    pl.kernel,
    out_type=jax.ShapeDtypeStruct((NPOOL2, 64), jnp.float32),
    mesh=_mesh(),
    compiler_params=_SC_PARAMS,
    scratch_types=[
        pltpu.VMEM((5, 32), jnp.int32),
        pltpu.VMEM((32, 64), jnp.float32),
        pltpu.SemaphoreType.DMA,
    ],
)
def _sc_scatter2(q_hbm, rankw_hbm, out_hbm, rkw_v, rowbuf_v, sem):
    c = lax.axis_index("c")
    s = lax.axis_index("s")
    wid = s * 2 + c
    pltpu.sync_copy(rankw_hbm.at[wid], rkw_v)
    for w in range(5):
        pltpu.async_copy(q_hbm.at[pl.ds(wid * 160 + w * 32, 32)], rowbuf_v,
                         sem).wait()
        pltpu.sync_copy(rowbuf_v, out_hbm.at[rkw_v.at[w]])


def _conv1_pre_body(x_ref, w_ref, deg_ref, y_ref, dinv_ref):
    deg = deg_ref[0, :] + deg_ref[1, :] + 1.0
    dinv = lax.rsqrt(deg)
    xw = jnp.dot(x_ref[...], w_ref[...], preferred_element_type=jnp.float32)
    y_ref[...] = xw * dinv[:, None]
    dinv_ref[...] = dinv


def _conv1_pre(xpad, W1p, deg_parts):
    bm = 2048
    return pl.pallas_call(
        _conv1_pre_body,
        out_shape=(
            jax.ShapeDtypeStruct((NPAD, 128), jnp.float32),
            jax.ShapeDtypeStruct((NPAD,), jnp.float32),
        ),
        grid=(NPAD // bm,),
        in_specs=[
            pl.BlockSpec((bm, 128), lambda i: (i, 0)),
            pl.BlockSpec((128, 128), lambda i: (0, 0)),
            pl.BlockSpec((2, bm), lambda i: (0, i)),
        ],
        out_specs=(
            pl.BlockSpec((bm, 128), lambda i: (i, 0)),
            pl.BlockSpec((bm,), lambda i: (i,)),
        ),
    )(xpad, W1p, deg_parts)


def _conv1_post_body(lo_ref, hi_ref, y_ref, dinv_ref, b_ref, p_ref, pn_ref,
                     hs_ref, s_ref):
    bm = hs_ref.shape[0]
    i = pl.program_id(0)
    dinv = dinv_ref[...][:, None]
    b = b_ref[...]
    h_lo = jnp.maximum((lo_ref[0] + lo_ref[1] + y_ref[:, :64]) * dinv + b[:, :64], 0.0)
    h_hi = jnp.maximum((hi_ref[0] + hi_ref[1] + y_ref[:, 64:]) * dinv + b[:, 64:], 0.0)
    pr = p_ref[...] * pn_ref[0, 0]
    raw = (jnp.dot(h_lo, pr[:64, :], preferred_element_type=jnp.float32)
           + jnp.dot(h_hi, pr[64:, :], preferred_element_type=jnp.float32))
    row = i * bm + lax.broadcasted_iota(jnp.int32, (bm, 1), 0)
    s1 = jnp.where(row < N, jnp.tanh(raw), -2.0)
    hs_ref[:, :64] = h_lo * s1
    hs_ref[:, 64:] = h_hi * s1
    s_ref[...] = s1[:, 0]


def _conv1_post(acc_lo, acc_hi, y1, dinv1, b1p, p1c, p1norm_inv):
    bm = 2048
    return pl.pallas_call(
        _conv1_post_body,
        out_shape=(
            jax.ShapeDtypeStruct((NPAD, 128), jnp.float32),
            jax.ShapeDtypeStruct((NPAD,), jnp.float32),
        ),
        grid=(NPAD // bm,),
        in_specs=[
            pl.BlockSpec((2, bm, 64), lambda i: (0, i, 0)),
            pl.BlockSpec((2, bm, 64), lambda i: (0, i, 0)),
            pl.BlockSpec((bm, 128), lambda i: (i, 0)),
            pl.BlockSpec((bm,), lambda i: (i,)),
            pl.BlockSpec((1, 128), lambda i: (0, 0)),
            pl.BlockSpec((128, 1), lambda i: (0, 0)),
            pl.BlockSpec((1, 1), lambda i: (0, 0), memory_space=pltpu.SMEM),
        ],
        out_specs=(
            pl.BlockSpec((bm, 128), lambda i: (i, 0)),
            pl.BlockSpec((bm,), lambda i: (i,)),
        ),
    )(acc_lo, acc_hi, y1, dinv1, b1p, p1c, p1norm_inv)


def _make_rank(n, bi, bj):
    def body(si_ref, sj_ref, o_ref):
        i = pl.program_id(0)
        j = pl.program_id(1)
        si = si_ref[0, :]
        sj = sj_ref[0, :]
        sic = si[:, None]
        sjr = sj[None, :]
        ig = i * bi + lax.broadcasted_iota(jnp.int32, (bi, bj), 0)
        jg = j * bj + lax.broadcasted_iota(jnp.int32, (bi, bj), 1)
        contrib = (sjr > sic) | ((sjr == sic) & (jg < ig))
        part = jnp.sum(jnp.where(contrib, 1.0, 0.0), axis=1)

        @pl.when(j == 0)
        def _():
            o_ref[...] = jnp.zeros_like(o_ref)

        o_ref[...] += part

    def rank(s):
        s2d = s.reshape(1, n)
        out = pl.pallas_call(
            body,
            out_shape=jax.ShapeDtypeStruct((n,), jnp.float32),
            grid=(n // bi, n // bj),
            in_specs=[
                pl.BlockSpec((1, bi), lambda i, j: (0, i)),
                pl.BlockSpec((1, bj), lambda i, j: (0, j)),
            ],
            out_specs=pl.BlockSpec((bi,), lambda i, j: (i,)),
        )(s2d, s2d)
        return out.astype(jnp.int32)

    return rank


_rank1 = _make_rank(NPAD, 512, 2048)
_rank2 = _make_rank(NPOOL2, 512, 1024)


def _conv2_pre_body(h1_ref, w_ref, deg_ref, z_ref, dinv_ref):
    bm = z_ref.shape[0]
    i = pl.program_id(0)
    deg = deg_ref[0, :] + deg_ref[1, :] + 1.0
    dinv = lax.rsqrt(deg)
    xw = jnp.dot(h1_ref[...], w_ref[...], preferred_element_type=jnp.float32)
    row = i * bm + lax.broadcasted_iota(jnp.int32, (bm, 1), 0)
    z_ref[...] = jnp.where(row < K1, xw * dinv[:, None], 0.0)
    dinv_ref[...] = dinv


def _conv2_pre(h1full, W2p, deg2_parts):
    bm = 1024
    return pl.pallas_call(
        _conv2_pre_body,
        out_shape=(
            jax.ShapeDtypeStruct((NPAD, 64), jnp.float32),
            jax.ShapeDtypeStruct((NPAD,), jnp.float32),
        ),
        grid=(NPAD // bm,),
        in_specs=[
            pl.BlockSpec((bm, 128), lambda i: (i, 0)),
            pl.BlockSpec((128, 64), lambda i: (0, 0)),
            pl.BlockSpec((2, bm), lambda i: (0, i)),
        ],
        out_specs=(
            pl.BlockSpec((bm, 64), lambda i: (i, 0)),
            pl.BlockSpec((bm,), lambda i: (i,)),
        ),
    )(h1full, W2p, deg2_parts)


def _conv2_post_body(acc_ref, z_ref, dinv_ref, b_ref, p_ref, pn_ref,
                     q_ref, s_ref):
    bm = q_ref.shape[0]
    i = pl.program_id(0)
    dinv = dinv_ref[...][:, None]
    h2 = jnp.maximum((acc_ref[0] + acc_ref[1] + z_ref[...]) * dinv + b_ref[...], 0.0)
    pr = p_ref[...] * pn_ref[0, 0]
    raw = jnp.dot(h2, pr, preferred_element_type=jnp.float32)
    row = i * bm + lax.broadcasted_iota(jnp.int32, (bm, 1), 0)
    s2 = jnp.where(row < K1, jnp.tanh(raw), -2.0)
    v = h2 * s2
    col = lax.broadcasted_iota(jnp.int32, (bm, 64), 1)
    cmask = col < H2_COLS
    m = jnp.max(jnp.where(cmask, v, -1e30), axis=1, keepdims=True)
    e = jnp.where(cmask, jnp.exp(v - m), 0.0)
    q_ref[...] = (v - m) - jnp.log(jnp.sum(e, axis=1, keepdims=True))
    s_ref[...] = s2[:, 0]


H2_COLS = 62


def _conv2_post(acc2, z2, dinv2, b2p, p2c, p2norm_inv):
    bm = 1024
    return pl.pallas_call(
        _conv2_post_body,
        out_shape=(
            jax.ShapeDtypeStruct((NPOOL2, 64), jnp.float32),
            jax.ShapeDtypeStruct((NPOOL2,), jnp.float32),
        ),
        grid=(NPOOL2 // bm,),
        in_specs=[
            pl.BlockSpec((2, bm, 64), lambda i: (0, i, 0)),
            pl.BlockSpec((bm, 64), lambda i: (i, 0)),
            pl.BlockSpec((bm,), lambda i: (i,)),
            pl.BlockSpec((1, 64), lambda i: (0, 0)),
            pl.BlockSpec((64, 1), lambda i: (0, 0)),
            pl.BlockSpec((1, 1), lambda i: (0, 0), memory_space=pltpu.SMEM),
        ],
        out_specs=(
            pl.BlockSpec((bm, 64), lambda i: (i, 0)),
            pl.BlockSpec((bm,), lambda i: (i,)),
        ),
    )(acc2, z2, dinv2, b2p, p2c, p2norm_inv)


def kernel(x, edge_index, W1, b1, p1, W2, b2, p2):
    f32 = jnp.float32
    src, dst = edge_index[0], edge_index[1]
    src_r = src.reshape(NW, WN, WL)
    dst_r = dst.reshape(NW, WN, WL)

    deg1_parts = _sc_hist(dst_r)
    xpad = jnp.pad(x, ((0, NPAD - N), (0, 0)))
    W1p = jnp.pad(W1, ((0, 0), (0, 128 - W1.shape[1])))
    y1, dinv1 = _conv1_pre(xpad, W1p, deg1_parts)
    acc_lo = _msgpass64(y1[:, :64], src_r, dst_r)
    acc_hi = _msgpass64(y1[:, 64:], src_r, dst_r)

    b1p = jnp.pad(b1, (0, 128 - b1.shape[0])).reshape(1, 128)
    p1c = jnp.pad(p1, (0, 128 - p1.shape[0])).reshape(128, 1)
    p1n = (1.0 / jnp.linalg.norm(p1)).reshape(1, 1)
    hs, s1 = _conv1_post(acc_lo, acc_hi, y1, dinv1, b1p, p1c, p1n)

    rank1 = _rank1(s1)
    h1full, gs_r, gd_r, deg2_parts = _sc_pool1(hs, rank1, src_r, dst_r)

    W2p = jnp.pad(W2, ((0, 2), (0, 64 - W2.shape[1])))
    z2, dinv2 = _conv2_pre(h1full, W2p, deg2_parts)
    acc2 = _msgpass64(z2, gs_r, gd_r)

    b2p = jnp.pad(b2, (0, 64 - b2.shape[0])).reshape(1, 64)
    p2c = jnp.pad(p2, (0, 64 - p2.shape[0])).reshape(64, 1)
    p2n = (1.0 / jnp.linalg.norm(p2)).reshape(1, 1)
    q, s2 = _conv2_post(acc2[:, :NPOOL2], z2[:NPOOL2], dinv2[:NPOOL2],
                        b2p, p2c, p2n)

    rank2 = _rank2(s2)
    rankw2 = rank2.reshape(NW, 5, 32)
    out_full = _sc_scatter2(q, rankw2)
    return out_full[:K2, :H2_COLS]


# speedup vs baseline: 34.9711x; 1.1252x over previous
import functools

import jax
import jax.numpy as jnp
import numpy as np
from jax import lax
from jax.experimental import pallas as pl
from jax.experimental.pallas import tpu as pltpu
from jax.experimental.pallas import tpu_sc as plsc

N = 10000
NPAD = 10240
E = 320000
NW = 32
WN = 125
WL = 80
K1 = 5000
K2 = 2500
NPOOL2 = 5120


def _mesh():
    return plsc.VectorSubcoreMesh(core_axis_name="c", subcore_axis_name="s")


_SC_PARAMS = pltpu.CompilerParams(use_tc_tiling_on_sc=False)
_SC_PARAMS_NL = pltpu.CompilerParams(use_tc_tiling_on_sc=False,
                                     needs_layout_passes=False)


@functools.partial(
    pl.kernel,
    out_type=jax.ShapeDtypeStruct((2, NPAD), jnp.float32),
    mesh=_mesh(),
    compiler_params=_SC_PARAMS,
    scratch_types=[
        pltpu.VMEM((WN, WL), jnp.int32),
        pltpu.VMEM((WL,), jnp.float32),
        pltpu.VMEM((NPAD // 16,), jnp.float32),
        pltpu.VMEM_SHARED((NPAD,), jnp.float32),
    ],
)
def _sc_hist(idx_hbm, out_hbm, idx_v, ones_v, z_v, acc_sh):
    c = lax.axis_index("c")
    s = lax.axis_index("s")
    wid = s * 2 + c
    sl = NPAD // 16

    def zbody(i, carry):
        z_v[pl.ds(i * 16, 16)] = jnp.zeros((16,), jnp.float32)
        return carry

    lax.fori_loop(0, sl // 16, zbody, 0)
    for u in range(WL // 16):
        ones_v[pl.ds(u * 16, 16)] = jnp.ones((16,), jnp.float32)
    pltpu.sync_copy(z_v, acc_sh.at[pl.ds(s * sl, sl)])
    plsc.subcore_barrier()

    pltpu.sync_copy(idx_hbm.at[wid], idx_v)

    def body(w, carry):
        pltpu.sync_copy(ones_v, acc_sh.at[idx_v.at[w]], add=True)
        return carry

    lax.fori_loop(0, WN, body, 0)
    plsc.subcore_barrier()
    pltpu.sync_copy(acc_sh.at[pl.ds(s * sl, sl)], out_hbm.at[c, pl.ds(s * sl, sl)])


def _make_msgpass(d_feat):
    @functools.partial(
        pl.kernel,
        out_type=jax.ShapeDtypeStruct((2, NPAD, d_feat), jnp.float32),
        mesh=_mesh(),
        compiler_params=_SC_PARAMS,
        scratch_types=[
            pltpu.VMEM((WN, WL), jnp.int32),
            pltpu.VMEM((WN, WL), jnp.int32),
            pltpu.VMEM((WL, d_feat), jnp.float32),
            pltpu.VMEM((WL, d_feat), jnp.float32),
            pltpu.VMEM((WL, d_feat), jnp.float32),
            pltpu.VMEM_SHARED((NPAD, d_feat), jnp.float32),
            pltpu.SemaphoreType.DMA,
            pltpu.SemaphoreType.DMA,
        ],
    )
    def msgpass(y_hbm, src_hbm, dst_hbm, out_hbm, sidx_v, didx_v, zb_v,
                rb0_v, rb1_v, acc_sh, sem0, sem1):
        c = lax.axis_index("c")
        s = lax.axis_index("s")
        wid = s * 2 + c
        rows = NPAD // 16

        def zbody(i, carry):
            for u in range(d_feat // 16):
                zb_v[i, pl.ds(u * 16, 16)] = jnp.zeros((16,), jnp.float32)
            return carry

        lax.fori_loop(0, WL, zbody, 0)
        for t in range(rows // WL):
            pltpu.sync_copy(zb_v, acc_sh.at[pl.ds(s * rows + t * WL, WL)])
        plsc.subcore_barrier()

        pltpu.sync_copy(src_hbm.at[wid], sidx_v)
        pltpu.sync_copy(dst_hbm.at[wid], didx_v)

        dummy = y_hbm.at[pl.ds(0, WL)]
        pltpu.async_copy(y_hbm.at[sidx_v.at[0]], rb0_v, sem0)

        def body(k, carry):
            w0 = 2 * k
            pltpu.make_async_copy(dummy, rb0_v, sem0).wait()
            pltpu.async_copy(y_hbm.at[sidx_v.at[w0 + 1]], rb1_v, sem1)
            pltpu.sync_copy(rb0_v, acc_sh.at[didx_v.at[w0]], add=True)
            pltpu.make_async_copy(dummy, rb1_v, sem1).wait()
            pltpu.async_copy(y_hbm.at[sidx_v.at[w0 + 2]], rb0_v, sem0)
            pltpu.sync_copy(rb1_v, acc_sh.at[didx_v.at[w0 + 1]], add=True)
            return carry

        lax.fori_loop(0, (WN - 1) // 2, body, 0)
        pltpu.make_async_copy(dummy, rb0_v, sem0).wait()
        pltpu.sync_copy(rb0_v, acc_sh.at[didx_v.at[WN - 1]], add=True)

        plsc.subcore_barrier()
        pltpu.sync_copy(acc_sh.at[pl.ds(s * rows, rows)],
                        out_hbm.at[c, pl.ds(s * rows, rows)])

    return msgpass


_msgpass64 = _make_msgpass(64)


@functools.partial(
    pl.kernel,
    out_type=(
        jax.ShapeDtypeStruct((NPAD, 128), jnp.float32),
        jax.ShapeDtypeStruct((NW, WN, WL), jnp.int32),
        jax.ShapeDtypeStruct((NW, WN, WL), jnp.int32),
        jax.ShapeDtypeStruct((2, NPAD), jnp.float32),
    ),
    mesh=_mesh(),
    compiler_params=_SC_PARAMS_NL,
    scratch_types=[
        pltpu.VMEM((NPAD,), jnp.int32),
        pltpu.VMEM((64,), jnp.int32),
        pltpu.VMEM((64, 128), jnp.float32),
        pltpu.VMEM((WN, WL), jnp.int32),
        pltpu.VMEM((WN, WL), jnp.int32),
        pltpu.VMEM((WN, WL), jnp.int32),
        pltpu.VMEM((WN, WL), jnp.int32),
        pltpu.VMEM((WN, WL), jnp.float32),
        pltpu.VMEM((NPAD // 16,), jnp.float32),
        pltpu.VMEM_SHARED((NPAD,), jnp.float32),
        pltpu.SemaphoreType.DMA,
    ],
)
def _sc_pool1(hs_hbm, rank_hbm, src_hbm, dst_hbm,
              h1_hbm, gs_hbm, gd_hbm, deg2_hbm,
              rank_v, rkw_v, rowbuf_v, sidx_v, didx_v, gsv, gdv, ksv, z_v,
              deg_sh, sem):
    c = lax.axis_index("c")
    s = lax.axis_index("s")
    wid = s * 2 + c
    sl = NPAD // 16

    def zbody(i, carry):
        z_v[pl.ds(i * 16, 16)] = jnp.zeros((16,), jnp.float32)
        return carry

    lax.fori_loop(0, sl // 16, zbody, 0)
    pltpu.sync_copy(z_v, deg_sh.at[pl.ds(s * sl, sl)])
    plsc.subcore_barrier()

    pltpu.sync_copy(rank_hbm, rank_v)
    for w in range(5):
        for u in range(4):
            rkw_v[pl.ds(u * 16, 16)] = rank_v[pl.ds(wid * 320 + w * 64 + u * 16, 16)]
        pltpu.async_copy(hs_hbm.at[pl.ds(wid * 320 + w * 64, 64)], rowbuf_v,
                         sem).wait()
        pltpu.sync_copy(rowbuf_v, h1_hbm.at[rkw_v])

    pltpu.sync_copy(src_hbm.at[wid], sidx_v)
    pltpu.sync_copy(dst_hbm.at[wid], didx_v)

    def body(w, carry):
        for u in range(WL // 16):
            s16 = sidx_v[w, pl.ds(u * 16, 16)]
            d16 = didx_v[w, pl.ds(u * 16, 16)]
            gs16 = plsc.load_gather(rank_v, [s16])
            gd16 = plsc.load_gather(rank_v, [d16])
            ks16 = jnp.where(gs16 < K1, 1.0, 0.0).astype(jnp.float32)
            gsv[w, pl.ds(u * 16, 16)] = gs16
            gdv[w, pl.ds(u * 16, 16)] = gd16
            ksv[w, pl.ds(u * 16, 16)] = ks16
        pltpu.sync_copy(ksv.at[w], deg_sh.at[gdv.at[w]], add=True)
        return carry

    lax.fori_loop(0, WN, body, 0)
    pltpu.sync_copy(gsv, gs_hbm.at[wid])
    pltpu.sync_copy(gdv, gd_hbm.at[wid])
    plsc.subcore_barrier()
    pltpu.sync_copy(deg_sh.at[pl.ds(s * sl, sl)], deg2_hbm.at[c, pl.ds(s * sl, sl)])


@functools.partial(
    pl.kernel,
    out_type=jax.ShapeDtypeStruct((NPOOL2, 64), jnp.float32),
    mesh=_mesh(),
    compiler_params=_SC_PARAMS,
    scratch_types=[
        pltpu.VMEM((5, 32), jnp.int32),
        pltpu.VMEM((32, 64), jnp.float32),
        pltpu.SemaphoreType.DMA,
    ],
)
def _sc_scatter2(q_hbm, rankw_hbm, out_hbm, rkw_v, rowbuf_v, sem):
    c = lax.axis_index("c")
    s = lax.axis_index("s")
    wid = s * 2 + c
    pltpu.sync_copy(rankw_hbm.at[wid], rkw_v)
    for w in range(5):
        pltpu.async_copy(q_hbm.at[pl.ds(wid * 160 + w * 32, 32)], rowbuf_v,
                         sem).wait()
        pltpu.sync_copy(rowbuf_v, out_hbm.at[rkw_v.at[w]])


def _conv1_pre_body(x_ref, w_ref, deg_ref, y_ref, dinv_ref):
    deg = deg_ref[0, :] + deg_ref[1, :] + 1.0
    dinv = lax.rsqrt(deg)
    xw = jnp.dot(x_ref[...], w_ref[...], preferred_element_type=jnp.float32)
    y_ref[...] = xw * dinv[:, None]
    dinv_ref[...] = dinv


def _conv1_pre(xpad, W1p, deg_parts):
    bm = 2048
    return pl.pallas_call(
        _conv1_pre_body,
        out_shape=(
            jax.ShapeDtypeStruct((NPAD, 128), jnp.float32),
            jax.ShapeDtypeStruct((NPAD,), jnp.float32),
        ),
        grid=(NPAD // bm,),
        in_specs=[
            pl.BlockSpec((bm, 128), lambda i: (i, 0)),
            pl.BlockSpec((128, 128), lambda i: (0, 0)),
            pl.BlockSpec((2, bm), lambda i: (0, i)),
        ],
        out_specs=(
            pl.BlockSpec((bm, 128), lambda i: (i, 0)),
            pl.BlockSpec((bm,), lambda i: (i,)),
        ),
    )(xpad, W1p, deg_parts)


def _conv1_post_body(lo_ref, hi_ref, y_ref, dinv_ref, b_ref, p_ref, pn_ref,
                     hs_ref, s_ref):
    bm = hs_ref.shape[0]
    i = pl.program_id(0)
    dinv = dinv_ref[...][:, None]
    b = b_ref[...]
    h_lo = jnp.maximum((lo_ref[0] + lo_ref[1] + y_ref[:, :64]) * dinv + b[:, :64], 0.0)
    h_hi = jnp.maximum((hi_ref[0] + hi_ref[1] + y_ref[:, 64:]) * dinv + b[:, 64:], 0.0)
    pr = p_ref[...] * pn_ref[0, 0]
    raw = (jnp.dot(h_lo, pr[:64, :], preferred_element_type=jnp.float32)
           + jnp.dot(h_hi, pr[64:, :], preferred_element_type=jnp.float32))
    row = i * bm + lax.broadcasted_iota(jnp.int32, (bm, 1), 0)
    s1 = jnp.where(row < N, jnp.tanh(raw), -2.0)
    hs_ref[:, :64] = h_lo * s1
    hs_ref[:, 64:] = h_hi * s1
    s_ref[...] = s1[:, 0]


def _conv1_post(acc_lo, acc_hi, y1, dinv1, b1p, p1c, p1norm_inv):
    bm = 2048
    return pl.pallas_call(
        _conv1_post_body,
        out_shape=(
            jax.ShapeDtypeStruct((NPAD, 128), jnp.float32),
            jax.ShapeDtypeStruct((NPAD,), jnp.float32),
        ),
        grid=(NPAD // bm,),
        in_specs=[
            pl.BlockSpec((2, bm, 64), lambda i: (0, i, 0)),
            pl.BlockSpec((2, bm, 64), lambda i: (0, i, 0)),
            pl.BlockSpec((bm, 128), lambda i: (i, 0)),
            pl.BlockSpec((bm,), lambda i: (i,)),
            pl.BlockSpec((1, 128), lambda i: (0, 0)),
            pl.BlockSpec((128, 1), lambda i: (0, 0)),
            pl.BlockSpec((1, 1), lambda i: (0, 0), memory_space=pltpu.SMEM),
        ],
        out_specs=(
            pl.BlockSpec((bm, 128), lambda i: (i, 0)),
            pl.BlockSpec((bm,), lambda i: (i,)),
        ),
    )(acc_lo, acc_hi, y1, dinv1, b1p, p1c, p1norm_inv)


def _make_rank(n, bi, bj):
    def body(si_ref, sj_ref, o_ref):
        i = pl.program_id(0)
        j = pl.program_id(1)
        si = si_ref[0, :]
        sj = sj_ref[0, :]
        sic = si[:, None]
        sjr = sj[None, :]
        ig = i * bi + lax.broadcasted_iota(jnp.int32, (bi, bj), 0)
        jg = j * bj + lax.broadcasted_iota(jnp.int32, (bi, bj), 1)
        contrib = (sjr > sic) | ((sjr == sic) & (jg < ig))
        part = jnp.sum(jnp.where(contrib, 1.0, 0.0), axis=1)

        @pl.when(j == 0)
        def _():
            o_ref[...] = jnp.zeros_like(o_ref)

        o_ref[...] += part

    def rank(s):
        s2d = s.reshape(1, n)
        out = pl.pallas_call(
            body,
            out_shape=jax.ShapeDtypeStruct((n,), jnp.float32),
            grid=(n // bi, n // bj),
            in_specs=[
                pl.BlockSpec((1, bi), lambda i, j: (0, i)),
                pl.BlockSpec((1, bj), lambda i, j: (0, j)),
            ],
            out_specs=pl.BlockSpec((bi,), lambda i, j: (i,)),
        )(s2d, s2d)
        return out.astype(jnp.int32)

    return rank


_rank1 = _make_rank(NPAD, 512, 2048)
_rank2 = _make_rank(NPOOL2, 512, 1024)


def _conv2_pre_body(h1_ref, w_ref, deg_ref, z_ref, dinv_ref):
    bm = z_ref.shape[0]
    i = pl.program_id(0)
    deg = deg_ref[0, :] + deg_ref[1, :] + 1.0
    dinv = lax.rsqrt(deg)
    xw = jnp.dot(h1_ref[...], w_ref[...], preferred_element_type=jnp.float32)
    row = i * bm + lax.broadcasted_iota(jnp.int32, (bm, 1), 0)
    z_ref[...] = jnp.where(row < K1, xw * dinv[:, None], 0.0)
    dinv_ref[...] = dinv


def _conv2_pre(h1full, W2p, deg2_parts):
    bm = 1024
    return pl.pallas_call(
        _conv2_pre_body,
        out_shape=(
            jax.ShapeDtypeStruct((NPAD, 64), jnp.float32),
            jax.ShapeDtypeStruct((NPAD,), jnp.float32),
        ),
        grid=(NPAD // bm,),
        in_specs=[
            pl.BlockSpec((bm, 128), lambda i: (i, 0)),
            pl.BlockSpec((128, 64), lambda i: (0, 0)),
            pl.BlockSpec((2, bm), lambda i: (0, i)),
        ],
        out_specs=(
            pl.BlockSpec((bm, 64), lambda i: (i, 0)),
            pl.BlockSpec((bm,), lambda i: (i,)),
        ),
    )(h1full, W2p, deg2_parts)


def _conv2_post_body(acc_ref, z_ref, dinv_ref, b_ref, p_ref, pn_ref,
                     q_ref, s_ref):
    bm = q_ref.shape[0]
    i = pl.program_id(0)
    dinv = dinv_ref[...][:, None]
    h2 = jnp.maximum((acc_ref[0] + acc_ref[1] + z_ref[...]) * dinv + b_ref[...], 0.0)
    pr = p_ref[...] * pn_ref[0, 0]
    raw = jnp.dot(h2, pr, preferred_element_type=jnp.float32)
    row = i * bm + lax.broadcasted_iota(jnp.int32, (bm, 1), 0)
    s2 = jnp.where(row < K1, jnp.tanh(raw), -2.0)
    v = h2 * s2
    col = lax.broadcasted_iota(jnp.int32, (bm, 64), 1)
    cmask = col < H2_COLS
    m = jnp.max(jnp.where(cmask, v, -1e30), axis=1, keepdims=True)
    e = jnp.where(cmask, jnp.exp(v - m), 0.0)
    q_ref[...] = (v - m) - jnp.log(jnp.sum(e, axis=1, keepdims=True))
    s_ref[...] = s2[:, 0]


H2_COLS = 62


def _conv2_post(acc2, z2, dinv2, b2p, p2c, p2norm_inv):
    bm = 1024
    return pl.pallas_call(
        _conv2_post_body,
        out_shape=(
            jax.ShapeDtypeStruct((NPOOL2, 64), jnp.float32),
            jax.ShapeDtypeStruct((NPOOL2,), jnp.float32),
        ),
        grid=(NPOOL2 // bm,),
        in_specs=[
            pl.BlockSpec((2, bm, 64), lambda i: (0, i, 0)),
            pl.BlockSpec((bm, 64), lambda i: (i, 0)),
            pl.BlockSpec((bm,), lambda i: (i,)),
            pl.BlockSpec((1, 64), lambda i: (0, 0)),
            pl.BlockSpec((64, 1), lambda i: (0, 0)),
            pl.BlockSpec((1, 1), lambda i: (0, 0), memory_space=pltpu.SMEM),
        ],
        out_specs=(
            pl.BlockSpec((bm, 64), lambda i: (i, 0)),
            pl.BlockSpec((bm,), lambda i: (i,)),
        ),
    )(acc2, z2, dinv2, b2p, p2c, p2norm_inv)


def kernel(x, edge_index, W1, b1, p1, W2, b2, p2):
    f32 = jnp.float32
    src, dst = edge_index[0], edge_index[1]
    src_r = src.reshape(NW, WN, WL)
    dst_r = dst.reshape(NW, WN, WL)

    deg1_parts = _sc_hist(dst_r)
    xpad = jnp.pad(x, ((0, NPAD - N), (0, 0)))
    W1p = jnp.pad(W1, ((0, 0), (0, 128 - W1.shape[1])))
    y1, dinv1 = _conv1_pre(xpad, W1p, deg1_parts)
    acc_lo = _msgpass64(y1[:, :64], src_r, dst_r)
    acc_hi = _msgpass64(y1[:, 64:], src_r, dst_r)

    b1p = jnp.pad(b1, (0, 128 - b1.shape[0])).reshape(1, 128)
    p1c = jnp.pad(p1, (0, 128 - p1.shape[0])).reshape(128, 1)
    p1n = (1.0 / jnp.linalg.norm(p1)).reshape(1, 1)
    hs, s1 = _conv1_post(acc_lo, acc_hi, y1, dinv1, b1p, p1c, p1n)

    rank1 = _rank1(s1)
    h1full, gs_r, gd_r, deg2_parts = _sc_pool1(hs, rank1, src_r, dst_r)

    W2p = jnp.pad(W2, ((0, 2), (0, 64 - W2.shape[1])))
    z2, dinv2 = _conv2_pre(h1full, W2p, deg2_parts)
    acc2 = _msgpass64(z2, gs_r, gd_r)

    b2p = jnp.pad(b2, (0, 64 - b2.shape[0])).reshape(1, 64)
    p2c = jnp.pad(p2, (0, 64 - p2.shape[0])).reshape(64, 1)
    p2n = (1.0 / jnp.linalg.norm(p2)).reshape(1, 1)
    q, s2 = _conv2_post(acc2[:, :NPOOL2], z2[:NPOOL2], dinv2[:NPOOL2],
                        b2p, p2c, p2n)

    rank2 = _rank2(s2)
    rankw2 = rank2.reshape(NW, 5, 32)
    out_full = _sc_scatter2(q, rankw2)
    return out_full[:K2, :H2_COLS]


# speedup vs baseline: 39.1466x; 1.1194x over previous
import functools

import jax
import jax.numpy as jnp
import numpy as np
from jax import lax
from jax.experimental import pallas as pl
from jax.experimental.pallas import tpu as pltpu
from jax.experimental.pallas import tpu_sc as plsc

N = 10000
NPAD = 10240
E = 320000
NW = 32
WN = 125
WL = 80
K1 = 5000
K2 = 2500
NPOOL2 = 5120


def _mesh():
    return plsc.VectorSubcoreMesh(core_axis_name="c", subcore_axis_name="s")


_SC_PARAMS = pltpu.CompilerParams(use_tc_tiling_on_sc=False)
_SC_PARAMS_NL = pltpu.CompilerParams(use_tc_tiling_on_sc=False,
                                     needs_layout_passes=False)


@functools.partial(
    pl.kernel,
    out_type=jax.ShapeDtypeStruct((2, NPAD), jnp.float32),
    mesh=_mesh(),
    compiler_params=_SC_PARAMS,
    scratch_types=[
        pltpu.VMEM((WN, WL), jnp.int32),
        pltpu.VMEM((WL,), jnp.float32),
        pltpu.VMEM((NPAD // 16,), jnp.float32),
        pltpu.VMEM_SHARED((NPAD,), jnp.float32),
    ],
)
def _sc_hist(idx_hbm, out_hbm, idx_v, ones_v, z_v, acc_sh):
    c = lax.axis_index("c")
    s = lax.axis_index("s")
    wid = s * 2 + c
    sl = NPAD // 16

    def zbody(i, carry):
        z_v[pl.ds(i * 16, 16)] = jnp.zeros((16,), jnp.float32)
        return carry

    lax.fori_loop(0, sl // 16, zbody, 0)
    for u in range(WL // 16):
        ones_v[pl.ds(u * 16, 16)] = jnp.ones((16,), jnp.float32)
    pltpu.sync_copy(z_v, acc_sh.at[pl.ds(s * sl, sl)])
    plsc.subcore_barrier()

    pltpu.sync_copy(idx_hbm.at[wid], idx_v)

    def body(w, carry):
        pltpu.sync_copy(ones_v, acc_sh.at[idx_v.at[w]], add=True)
        return carry

    lax.fori_loop(0, WN, body, 0)
    plsc.subcore_barrier()
    pltpu.sync_copy(acc_sh.at[pl.ds(s * sl, sl)], out_hbm.at[c, pl.ds(s * sl, sl)])


def _make_msgpass(d_feat, wn, wl):
    @functools.partial(
        pl.kernel,
        out_type=jax.ShapeDtypeStruct((2, NPAD, d_feat), jnp.float32),
        mesh=_mesh(),
        compiler_params=_SC_PARAMS,
        scratch_types=[
            pltpu.VMEM((wn, wl), jnp.int32),
            pltpu.VMEM((wn, wl), jnp.int32),
            pltpu.VMEM((80, d_feat), jnp.float32),
            pltpu.VMEM((wl, d_feat), jnp.float32),
            pltpu.VMEM((wl, d_feat), jnp.float32),
            pltpu.VMEM_SHARED((NPAD, d_feat), jnp.float32),
            pltpu.SemaphoreType.DMA,
            pltpu.SemaphoreType.DMA,
        ],
    )
    def msgpass(y_hbm, src_hbm, dst_hbm, out_hbm, sidx_v, didx_v, zb_v,
                rb0_v, rb1_v, acc_sh, sem0, sem1):
        c = lax.axis_index("c")
        s = lax.axis_index("s")
        wid = s * 2 + c
        rows = NPAD // 16

        def zbody(i, carry):
            for u in range(d_feat // 16):
                zb_v[i, pl.ds(u * 16, 16)] = jnp.zeros((16,), jnp.float32)
            return carry

        lax.fori_loop(0, 80, zbody, 0)
        for t in range(rows // 80):
            pltpu.sync_copy(zb_v, acc_sh.at[pl.ds(s * rows + t * 80, 80)])
        plsc.subcore_barrier()

        pltpu.sync_copy(src_hbm.at[wid], sidx_v)
        pltpu.sync_copy(dst_hbm.at[wid], didx_v)

        dummy = y_hbm.at[pl.ds(0, wl)]
        half = (wn - 1) // 2
        pltpu.async_copy(y_hbm.at[sidx_v.at[0]], rb0_v, sem0)

        def body(k, carry):
            w0 = 2 * k
            pltpu.make_async_copy(dummy, rb0_v, sem0).wait()
            pltpu.async_copy(y_hbm.at[sidx_v.at[w0 + 1]], rb1_v, sem1)
            pltpu.sync_copy(rb0_v, acc_sh.at[didx_v.at[w0]], add=True)
            pltpu.make_async_copy(dummy, rb1_v, sem1).wait()
            pltpu.async_copy(y_hbm.at[sidx_v.at[w0 + 2]], rb0_v, sem0)
            pltpu.sync_copy(rb1_v, acc_sh.at[didx_v.at[w0 + 1]], add=True)
            return carry

        lax.fori_loop(0, half, body, 0)
        pltpu.make_async_copy(dummy, rb0_v, sem0).wait()
        if wn % 2 == 0:
            pltpu.async_copy(y_hbm.at[sidx_v.at[wn - 1]], rb1_v, sem1)
            pltpu.sync_copy(rb0_v, acc_sh.at[didx_v.at[wn - 2]], add=True)
            pltpu.make_async_copy(dummy, rb1_v, sem1).wait()
            pltpu.sync_copy(rb1_v, acc_sh.at[didx_v.at[wn - 1]], add=True)
        else:
            pltpu.sync_copy(rb0_v, acc_sh.at[didx_v.at[wn - 1]], add=True)

        plsc.subcore_barrier()
        pltpu.sync_copy(acc_sh.at[pl.ds(s * rows, rows)],
                        out_hbm.at[c, pl.ds(s * rows, rows)])

    return msgpass


_msgpass64_c1 = _make_msgpass(64, WN, WL)
_msgpass64_c2 = _msgpass64_c1


@functools.partial(
    pl.kernel,
    out_type=(
        jax.ShapeDtypeStruct((NPAD, 128), jnp.float32),
        jax.ShapeDtypeStruct((NW, WN, WL), jnp.int32),
        jax.ShapeDtypeStruct((NW, WN, WL), jnp.int32),
        jax.ShapeDtypeStruct((2, NPAD), jnp.float32),
    ),
    mesh=_mesh(),
    compiler_params=_SC_PARAMS_NL,
    scratch_types=[
        pltpu.VMEM((NPAD,), jnp.int32),
        pltpu.VMEM((64,), jnp.int32),
        pltpu.VMEM((64, 128), jnp.float32),
        pltpu.VMEM((WN, WL), jnp.int32),
        pltpu.VMEM((WN, WL), jnp.int32),
        pltpu.VMEM((WN, WL), jnp.int32),
        pltpu.VMEM((WN, WL), jnp.int32),
        pltpu.VMEM((WN, WL), jnp.float32),
        pltpu.VMEM((NPAD // 16,), jnp.float32),
        pltpu.VMEM_SHARED((NPAD,), jnp.float32),
        pltpu.SemaphoreType.DMA,
    ],
)
def _sc_pool1(hs_hbm, rank_hbm, src_hbm, dst_hbm,
              h1_hbm, gs_hbm, gd_hbm, deg2_hbm,
              rank_v, rkw_v, rowbuf_v, sidx_v, didx_v, gsv, gdv, ksv, z_v,
              deg_sh, sem):
    c = lax.axis_index("c")
    s = lax.axis_index("s")
    wid = s * 2 + c
    sl = NPAD // 16

    def zbody(i, carry):
        z_v[pl.ds(i * 16, 16)] = jnp.zeros((16,), jnp.float32)
        return carry

    lax.fori_loop(0, sl // 16, zbody, 0)
    pltpu.sync_copy(z_v, deg_sh.at[pl.ds(s * sl, sl)])
    plsc.subcore_barrier()

    pltpu.sync_copy(rank_hbm, rank_v)
    for w in range(5):
        for u in range(4):
            rkw_v[pl.ds(u * 16, 16)] = rank_v[pl.ds(wid * 320 + w * 64 + u * 16, 16)]
        pltpu.async_copy(hs_hbm.at[pl.ds(wid * 320 + w * 64, 64)], rowbuf_v,
                         sem).wait()
        pltpu.sync_copy(rowbuf_v, h1_hbm.at[rkw_v])

    pltpu.sync_copy(src_hbm.at[wid], sidx_v)
    pltpu.sync_copy(dst_hbm.at[wid], didx_v)

    def body(w, carry):
        for u in range(WL // 16):
            s16 = sidx_v[w, pl.ds(u * 16, 16)]
            d16 = didx_v[w, pl.ds(u * 16, 16)]
            gs16 = plsc.load_gather(rank_v, [s16])
            gd16 = plsc.load_gather(rank_v, [d16])
            ks16 = jnp.where(gs16 < K1, 1.0, 0.0).astype(jnp.float32)
            gsv[w, pl.ds(u * 16, 16)] = gs16
            gdv[w, pl.ds(u * 16, 16)] = gd16
            ksv[w, pl.ds(u * 16, 16)] = ks16
        pltpu.sync_copy(ksv.at[w], deg_sh.at[gdv.at[w]], add=True)
        return carry

    lax.fori_loop(0, WN, body, 0)
    pltpu.sync_copy(gsv, gs_hbm.at[wid])
    pltpu.sync_copy(gdv, gd_hbm.at[wid])
    plsc.subcore_barrier()
    pltpu.sync_copy(deg_sh.at[pl.ds(s * sl, sl)], deg2_hbm.at[c, pl.ds(s * sl, sl)])


@functools.partial(
    pl.kernel,
    out_type=jax.ShapeDtypeStruct((NPOOL2, 64), jnp.float32),
    mesh=_mesh(),
    compiler_params=_SC_PARAMS,
    scratch_types=[
        pltpu.VMEM((5, 32), jnp.int32),
        pltpu.VMEM((32, 64), jnp.float32),
        pltpu.SemaphoreType.DMA,
    ],
)
def _sc_scatter2(q_hbm, rankw_hbm, out_hbm, rkw_v, rowbuf_v, sem):
    c = lax.axis_index("c")
    s = lax.axis_index("s")
    wid = s * 2 + c
    pltpu.sync_copy(rankw_hbm.at[wid], rkw_v)
    for w in range(5):
        pltpu.async_copy(q_hbm.at[pl.ds(wid * 160 + w * 32, 32)], rowbuf_v,
                         sem).wait()
        pltpu.sync_copy(rowbuf_v, out_hbm.at[rkw_v.at[w]])


def _conv1_pre_body(x_ref, w_ref, deg_ref, y_ref, dinv_ref):
    deg = deg_ref[0, :] + deg_ref[1, :] + 1.0
    dinv = lax.rsqrt(deg)
    xw = jnp.dot(x_ref[...], w_ref[...], preferred_element_type=jnp.float32)
    y_ref[...] = xw * dinv[:, None]
    dinv_ref[...] = dinv


def _conv1_pre(xpad, W1p, deg_parts):
    bm = 2048
    return pl.pallas_call(
        _conv1_pre_body,
        out_shape=(
            jax.ShapeDtypeStruct((NPAD, 128), jnp.float32),
            jax.ShapeDtypeStruct((NPAD,), jnp.float32),
        ),
        grid=(NPAD // bm,),
        in_specs=[
            pl.BlockSpec((bm, 128), lambda i: (i, 0)),
            pl.BlockSpec((128, 128), lambda i: (0, 0)),
            pl.BlockSpec((2, bm), lambda i: (0, i)),
        ],
        out_specs=(
            pl.BlockSpec((bm, 128), lambda i: (i, 0)),
            pl.BlockSpec((bm,), lambda i: (i,)),
        ),
    )(xpad, W1p, deg_parts)


def _conv1_post_body(lo_ref, hi_ref, y_ref, dinv_ref, b_ref, p_ref, pn_ref,
                     hs_ref, s_ref):
    bm = hs_ref.shape[0]
    i = pl.program_id(0)
    dinv = dinv_ref[...][:, None]
    b = b_ref[...]
    h_lo = jnp.maximum((lo_ref[0] + lo_ref[1] + y_ref[:, :64]) * dinv + b[:, :64], 0.0)
    h_hi = jnp.maximum((hi_ref[0] + hi_ref[1] + y_ref[:, 64:]) * dinv + b[:, 64:], 0.0)
    pr = p_ref[...] * pn_ref[0, 0]
    raw = (jnp.dot(h_lo, pr[:64, :], preferred_element_type=jnp.float32)
           + jnp.dot(h_hi, pr[64:, :], preferred_element_type=jnp.float32))
    row = i * bm + lax.broadcasted_iota(jnp.int32, (bm, 1), 0)
    s1 = jnp.where(row < N, jnp.tanh(raw), -2.0)
    hs_ref[:, :64] = h_lo * s1
    hs_ref[:, 64:] = h_hi * s1
    s_ref[...] = s1[:, 0]


def _conv1_post(acc_lo, acc_hi, y1, dinv1, b1p, p1c, p1norm_inv):
    bm = 2048
    return pl.pallas_call(
        _conv1_post_body,
        out_shape=(
            jax.ShapeDtypeStruct((NPAD, 128), jnp.float32),
            jax.ShapeDtypeStruct((NPAD,), jnp.float32),
        ),
        grid=(NPAD // bm,),
        in_specs=[
            pl.BlockSpec((2, bm, 64), lambda i: (0, i, 0)),
            pl.BlockSpec((2, bm, 64), lambda i: (0, i, 0)),
            pl.BlockSpec((bm, 128), lambda i: (i, 0)),
            pl.BlockSpec((bm,), lambda i: (i,)),
            pl.BlockSpec((1, 128), lambda i: (0, 0)),
            pl.BlockSpec((128, 1), lambda i: (0, 0)),
            pl.BlockSpec((1, 1), lambda i: (0, 0), memory_space=pltpu.SMEM),
        ],
        out_specs=(
            pl.BlockSpec((bm, 128), lambda i: (i, 0)),
            pl.BlockSpec((bm,), lambda i: (i,)),
        ),
    )(acc_lo, acc_hi, y1, dinv1, b1p, p1c, p1norm_inv)


def _cast_body(a_ref, o_ref):
    o_ref[...] = a_ref[...].astype(jnp.int32)


def _make_rank(n, b):
    def body(si_ref, sj_ref, o1_ref):
        i = pl.program_id(0)
        j = pl.program_id(1)

        @pl.when(j == 0)
        def _():
            o1_ref[...] = jnp.zeros_like(o1_ref)

        @pl.when(j > i)
        def _():
            sic = si_ref[0, :][:, None]
            sjr = sj_ref[0, :][None, :]
            o1_ref[...] += jnp.sum(jnp.where(sjr > sic, 1.0, 0.0), axis=1)

        @pl.when(j < i)
        def _():
            sic = si_ref[0, :][:, None]
            sjr = sj_ref[0, :][None, :]
            o1_ref[...] += jnp.sum(jnp.where(sjr >= sic, 1.0, 0.0), axis=1)

        @pl.when(j == i)
        def _():
            si = si_ref[0, :]
            sic = si[:, None]
            sjr = si[None, :]
            il = lax.broadcasted_iota(jnp.int32, (b, b), 0)
            jl = lax.broadcasted_iota(jnp.int32, (b, b), 1)
            contrib = (sjr > sic) | ((sjr == sic) & (jl < il))
            o1_ref[...] += jnp.sum(jnp.where(contrib, 1.0, 0.0), axis=1)

    def rank(s):
        s2d = s.reshape(1, n)
        o1 = pl.pallas_call(
            body,
            out_shape=jax.ShapeDtypeStruct((n,), jnp.float32),
            grid=(n // b, n // b),
            in_specs=[
                pl.BlockSpec((1, b), lambda i, j: (0, i)),
                pl.BlockSpec((1, b), lambda i, j: (0, j)),
            ],
            out_specs=pl.BlockSpec((b,), lambda i, j: (i,)),
        )(s2d, s2d)
        return pl.pallas_call(
            _cast_body,
            out_shape=jax.ShapeDtypeStruct((n,), jnp.int32),
        )(o1)

    return rank


_rank1 = _make_rank(NPAD, 1024)
_rank2 = _make_rank(NPOOL2, 1024)


def _conv2_pre_body(h1_ref, w_ref, deg_ref, z_ref, dinv_ref):
    bm = z_ref.shape[0]
    i = pl.program_id(0)
    deg = deg_ref[0, :] + deg_ref[1, :] + 1.0
    dinv = lax.rsqrt(deg)
    xw = jnp.dot(h1_ref[...], w_ref[...], preferred_element_type=jnp.float32)
    row = i * bm + lax.broadcasted_iota(jnp.int32, (bm, 1), 0)
    z_ref[...] = jnp.where(row < K1, xw * dinv[:, None], 0.0)
    dinv_ref[...] = dinv


def _conv2_pre(h1full, W2p, deg2_parts):
    bm = 1024
    return pl.pallas_call(
        _conv2_pre_body,
        out_shape=(
            jax.ShapeDtypeStruct((NPAD, 64), jnp.float32),
            jax.ShapeDtypeStruct((NPAD,), jnp.float32),
        ),
        grid=(NPAD // bm,),
        in_specs=[
            pl.BlockSpec((bm, 128), lambda i: (i, 0)),
            pl.BlockSpec((128, 64), lambda i: (0, 0)),
            pl.BlockSpec((2, bm), lambda i: (0, i)),
        ],
        out_specs=(
            pl.BlockSpec((bm, 64), lambda i: (i, 0)),
            pl.BlockSpec((bm,), lambda i: (i,)),
        ),
    )(h1full, W2p, deg2_parts)


def _conv2_post_body(acc_ref, z_ref, dinv_ref, b_ref, p_ref, pn_ref,
                     q_ref, s_ref):
    bm = q_ref.shape[0]
    i = pl.program_id(0)
    dinv = dinv_ref[...][:, None]
    h2 = jnp.maximum((acc_ref[0] + acc_ref[1] + z_ref[...]) * dinv + b_ref[...], 0.0)
    pr = p_ref[...] * pn_ref[0, 0]
    raw = jnp.dot(h2, pr, preferred_element_type=jnp.float32)
    row = i * bm + lax.broadcasted_iota(jnp.int32, (bm, 1), 0)
    s2 = jnp.where(row < K1, jnp.tanh(raw), -2.0)
    v = h2 * s2
    col = lax.broadcasted_iota(jnp.int32, (bm, 64), 1)
    cmask = col < H2_COLS
    m = jnp.max(jnp.where(cmask, v, -1e30), axis=1, keepdims=True)
    e = jnp.where(cmask, jnp.exp(v - m), 0.0)
    q_ref[...] = (v - m) - jnp.log(jnp.sum(e, axis=1, keepdims=True))
    s_ref[...] = s2[:, 0]


H2_COLS = 62


def _conv2_post(acc2, z2, dinv2, b2p, p2c, p2norm_inv):
    bm = 1024
    return pl.pallas_call(
        _conv2_post_body,
        out_shape=(
            jax.ShapeDtypeStruct((NPOOL2, 64), jnp.float32),
            jax.ShapeDtypeStruct((NPOOL2,), jnp.float32),
        ),
        grid=(NPOOL2 // bm,),
        in_specs=[
            pl.BlockSpec((2, bm, 64), lambda i: (0, i, 0)),
            pl.BlockSpec((bm, 64), lambda i: (i, 0)),
            pl.BlockSpec((bm,), lambda i: (i,)),
            pl.BlockSpec((1, 64), lambda i: (0, 0)),
            pl.BlockSpec((64, 1), lambda i: (0, 0)),
            pl.BlockSpec((1, 1), lambda i: (0, 0), memory_space=pltpu.SMEM),
        ],
        out_specs=(
            pl.BlockSpec((bm, 64), lambda i: (i, 0)),
            pl.BlockSpec((bm,), lambda i: (i,)),
        ),
    )(acc2, z2, dinv2, b2p, p2c, p2norm_inv)


def kernel(x, edge_index, W1, b1, p1, W2, b2, p2):
    f32 = jnp.float32
    src, dst = edge_index[0], edge_index[1]
    src_r = src.reshape(NW, WN, WL)
    dst_r = dst.reshape(NW, WN, WL)

    src_c1 = src_r
    dst_c1 = dst_r

    deg1_parts = _sc_hist(dst_r)
    xpad = jnp.pad(x, ((0, NPAD - N), (0, 0)))
    W1p = jnp.pad(W1, ((0, 0), (0, 128 - W1.shape[1])))
    y1, dinv1 = _conv1_pre(xpad, W1p, deg1_parts)
    acc_lo = _msgpass64_c1(y1[:, :64], src_c1, dst_c1)
    acc_hi = _msgpass64_c1(y1[:, 64:], src_c1, dst_c1)

    b1p = jnp.pad(b1, (0, 128 - b1.shape[0])).reshape(1, 128)
    p1c = jnp.pad(p1, (0, 128 - p1.shape[0])).reshape(128, 1)
    p1n = (1.0 / jnp.linalg.norm(p1)).reshape(1, 1)
    hs, s1 = _conv1_post(acc_lo, acc_hi, y1, dinv1, b1p, p1c, p1n)

    rank1 = _rank1(s1)
    h1full, gs_r, gd_r, deg2_parts = _sc_pool1(hs, rank1, src_r, dst_r)

    W2p = jnp.pad(W2, ((0, 2), (0, 64 - W2.shape[1])))
    z2, dinv2 = _conv2_pre(h1full, W2p, deg2_parts)
    acc2 = _msgpass64_c2(z2, gs_r, gd_r)

    b2p = jnp.pad(b2, (0, 64 - b2.shape[0])).reshape(1, 64)
    p2c = jnp.pad(p2, (0, 64 - p2.shape[0])).reshape(64, 1)
    p2n = (1.0 / jnp.linalg.norm(p2)).reshape(1, 1)
    q, s2 = _conv2_post(acc2[:, :NPOOL2], z2[:NPOOL2], dinv2[:NPOOL2],
                        b2p, p2c, p2n)

    rank2 = _rank2(s2)
    rankw2 = rank2.reshape(NW, 5, 32)
    out_full = _sc_scatter2(q, rankw2)
    return out_full[:K2, :H2_COLS]


# speedup vs baseline: 54.4476x; 1.3909x over previous
import functools

import jax
import jax.numpy as jnp
import numpy as np
from jax import lax
from jax.experimental import pallas as pl
from jax.experimental.pallas import tpu as pltpu
from jax.experimental.pallas import tpu_sc as plsc

N = 10000
NPAD = 10240
E = 320000
NW = 32
WN = 125
WL = 80
K1 = 5000
K2 = 2500
NPOOL2 = 5120


def _mesh():
    return plsc.VectorSubcoreMesh(core_axis_name="c", subcore_axis_name="s")


_SC_PARAMS = pltpu.CompilerParams(use_tc_tiling_on_sc=False)
_SC_PARAMS_NL = pltpu.CompilerParams(use_tc_tiling_on_sc=False,
                                     needs_layout_passes=False)


@functools.partial(
    pl.kernel,
    out_type=jax.ShapeDtypeStruct((2, NPAD), jnp.float32),
    mesh=_mesh(),
    compiler_params=_SC_PARAMS,
    scratch_types=[
        pltpu.VMEM((WN, WL), jnp.int32),
        pltpu.VMEM((WL,), jnp.float32),
        pltpu.VMEM((NPAD // 16,), jnp.float32),
        pltpu.VMEM_SHARED((NPAD,), jnp.float32),
    ],
)
def _sc_hist(idx_hbm, out_hbm, idx_v, ones_v, z_v, acc_sh):
    c = lax.axis_index("c")
    s = lax.axis_index("s")
    wid = s * 2 + c
    sl = NPAD // 16

    def zbody(i, carry):
        z_v[pl.ds(i * 16, 16)] = jnp.zeros((16,), jnp.float32)
        return carry

    lax.fori_loop(0, sl // 16, zbody, 0)
    for u in range(WL // 16):
        ones_v[pl.ds(u * 16, 16)] = jnp.ones((16,), jnp.float32)
    pltpu.sync_copy(z_v, acc_sh.at[pl.ds(s * sl, sl)])
    plsc.subcore_barrier()

    pltpu.sync_copy(idx_hbm.at[wid], idx_v)

    def body(w, carry):
        pltpu.sync_copy(ones_v, acc_sh.at[idx_v.at[w]], add=True)
        return carry

    lax.fori_loop(0, WN, body, 0)
    plsc.subcore_barrier()
    pltpu.sync_copy(acc_sh.at[pl.ds(s * sl, sl)], out_hbm.at[c, pl.ds(s * sl, sl)])


def _make_msgpass(d_feat, wn, wl):
    @functools.partial(
        pl.kernel,
        out_type=jax.ShapeDtypeStruct((2, NPAD, d_feat), jnp.float32),
        mesh=_mesh(),
        compiler_params=_SC_PARAMS,
        scratch_types=[
            pltpu.VMEM((wn, wl), jnp.int32),
            pltpu.VMEM((wn, wl), jnp.int32),
            pltpu.VMEM((80, d_feat), jnp.float32),
            [pltpu.VMEM((wl, d_feat), jnp.float32) for _ in range(4)],
            pltpu.VMEM_SHARED((NPAD, d_feat), jnp.float32),
            [pltpu.SemaphoreType.DMA for _ in range(4)],
        ],
    )
    def msgpass(y_hbm, src_hbm, dst_hbm, out_hbm, sidx_v, didx_v, zb_v,
                rbufs, acc_sh, sems):
        c = lax.axis_index("c")
        s = lax.axis_index("s")
        wid = s * 2 + c
        rows = NPAD // 16

        def zbody(i, carry):
            for u in range(d_feat // 16):
                zb_v[i, pl.ds(u * 16, 16)] = jnp.zeros((16,), jnp.float32)
            return carry

        lax.fori_loop(0, 80, zbody, 0)
        for t in range(rows // 80):
            pltpu.sync_copy(zb_v, acc_sh.at[pl.ds(s * rows + t * 80, 80)])
        plsc.subcore_barrier()

        pltpu.sync_copy(src_hbm.at[wid], sidx_v)
        pltpu.sync_copy(dst_hbm.at[wid], didx_v)

        nbuf = 4
        dummy = y_hbm.at[pl.ds(0, wl)]
        for b in range(nbuf - 1):
            pltpu.async_copy(y_hbm.at[sidx_v.at[b]], rbufs[b], sems[b])

        def body(k, carry):
            w0 = nbuf * k
            for u in range(nbuf):
                w = w0 + u
                pltpu.make_async_copy(dummy, rbufs[u], sems[u]).wait()
                nxt = w + nbuf - 1
                bn = (u + nbuf - 1) % nbuf

                @pl.when(nxt < wn)
                def _():
                    pltpu.async_copy(y_hbm.at[sidx_v.at[nxt]], rbufs[bn],
                                     sems[bn])

                pltpu.sync_copy(rbufs[u], acc_sh.at[didx_v.at[w]], add=True)
            return carry

        lax.fori_loop(0, wn // nbuf, body, 0)
        for u in range(wn % nbuf):
            w = (wn // nbuf) * nbuf + u
            pltpu.make_async_copy(dummy, rbufs[w % nbuf], sems[w % nbuf]).wait()
            pltpu.sync_copy(rbufs[w % nbuf], acc_sh.at[didx_v.at[w]], add=True)

        plsc.subcore_barrier()
        pltpu.sync_copy(acc_sh.at[pl.ds(s * rows, rows)],
                        out_hbm.at[c, pl.ds(s * rows, rows)])

    return msgpass


_msgpass64_c1 = _make_msgpass(64, WN, WL)
_msgpass64_c2 = _msgpass64_c1


@functools.partial(
    pl.kernel,
    out_type=(
        jax.ShapeDtypeStruct((NPAD, 128), jnp.float32),
        jax.ShapeDtypeStruct((NW, WN, WL), jnp.int32),
        jax.ShapeDtypeStruct((NW, WN, WL), jnp.int32),
        jax.ShapeDtypeStruct((2, NPAD), jnp.float32),
    ),
    mesh=_mesh(),
    compiler_params=_SC_PARAMS_NL,
    scratch_types=[
        pltpu.VMEM((NPAD,), jnp.int32),
        pltpu.VMEM((64,), jnp.int32),
        pltpu.VMEM((64, 128), jnp.float32),
        pltpu.VMEM((WN, WL), jnp.int32),
        pltpu.VMEM((WN, WL), jnp.int32),
        pltpu.VMEM((WN, WL), jnp.int32),
        pltpu.VMEM((WN, WL), jnp.int32),
        pltpu.VMEM((WN, WL), jnp.float32),
        pltpu.VMEM((NPAD // 16,), jnp.float32),
        pltpu.VMEM_SHARED((NPAD,), jnp.float32),
        pltpu.SemaphoreType.DMA,
    ],
)
def _sc_pool1(hs_hbm, rank_hbm, src_hbm, dst_hbm,
              h1_hbm, gs_hbm, gd_hbm, deg2_hbm,
              rank_v, rkw_v, rowbuf_v, sidx_v, didx_v, gsv, gdv, ksv, z_v,
              deg_sh, sem):
    c = lax.axis_index("c")
    s = lax.axis_index("s")
    wid = s * 2 + c
    sl = NPAD // 16

    def zbody(i, carry):
        z_v[pl.ds(i * 16, 16)] = jnp.zeros((16,), jnp.float32)
        return carry

    lax.fori_loop(0, sl // 16, zbody, 0)
    pltpu.sync_copy(z_v, deg_sh.at[pl.ds(s * sl, sl)])
    plsc.subcore_barrier()

    pltpu.sync_copy(rank_hbm, rank_v)
    for w in range(5):
        for u in range(4):
            rkw_v[pl.ds(u * 16, 16)] = rank_v[pl.ds(wid * 320 + w * 64 + u * 16, 16)]
        pltpu.async_copy(hs_hbm.at[pl.ds(wid * 320 + w * 64, 64)], rowbuf_v,
                         sem).wait()
        pltpu.sync_copy(rowbuf_v, h1_hbm.at[rkw_v])

    pltpu.sync_copy(src_hbm.at[wid], sidx_v)
    pltpu.sync_copy(dst_hbm.at[wid], didx_v)

    def body(w, carry):
        for u in range(WL // 16):
            s16 = sidx_v[w, pl.ds(u * 16, 16)]
            d16 = didx_v[w, pl.ds(u * 16, 16)]
            gs16 = plsc.load_gather(rank_v, [s16])
            gd16 = plsc.load_gather(rank_v, [d16])
            ks16 = jnp.where(gs16 < K1, 1.0, 0.0).astype(jnp.float32)
            gsv[w, pl.ds(u * 16, 16)] = gs16
            gdv[w, pl.ds(u * 16, 16)] = gd16
            ksv[w, pl.ds(u * 16, 16)] = ks16
        pltpu.sync_copy(ksv.at[w], deg_sh.at[gdv.at[w]], add=True)
        return carry

    lax.fori_loop(0, WN, body, 0)
    pltpu.sync_copy(gsv, gs_hbm.at[wid])
    pltpu.sync_copy(gdv, gd_hbm.at[wid])
    plsc.subcore_barrier()
    pltpu.sync_copy(deg_sh.at[pl.ds(s * sl, sl)], deg2_hbm.at[c, pl.ds(s * sl, sl)])


@functools.partial(
    pl.kernel,
    out_type=jax.ShapeDtypeStruct((NPOOL2, 64), jnp.float32),
    mesh=_mesh(),
    compiler_params=_SC_PARAMS,
    scratch_types=[
        pltpu.VMEM((5, 32), jnp.int32),
        pltpu.VMEM((32, 64), jnp.float32),
        pltpu.SemaphoreType.DMA,
    ],
)
def _sc_scatter2(q_hbm, rankw_hbm, out_hbm, rkw_v, rowbuf_v, sem):
    c = lax.axis_index("c")
    s = lax.axis_index("s")
    wid = s * 2 + c
    pltpu.sync_copy(rankw_hbm.at[wid], rkw_v)
    for w in range(5):
        pltpu.async_copy(q_hbm.at[pl.ds(wid * 160 + w * 32, 32)], rowbuf_v,
                         sem).wait()
        pltpu.sync_copy(rowbuf_v, out_hbm.at[rkw_v.at[w]])


def _conv1_pre_body(x_ref, w_ref, deg_ref, y_ref, dinv_ref):
    deg = deg_ref[0, :] + deg_ref[1, :] + 1.0
    dinv = lax.rsqrt(deg)
    xw = jnp.dot(x_ref[...], w_ref[...], preferred_element_type=jnp.float32)
    y_ref[...] = xw * dinv[:, None]
    dinv_ref[...] = dinv


def _conv1_pre(xpad, W1p, deg_parts):
    bm = 2048
    return pl.pallas_call(
        _conv1_pre_body,
        out_shape=(
            jax.ShapeDtypeStruct((NPAD, 128), jnp.float32),
            jax.ShapeDtypeStruct((NPAD,), jnp.float32),
        ),
        grid=(NPAD // bm,),
        in_specs=[
            pl.BlockSpec((bm, 128), lambda i: (i, 0)),
            pl.BlockSpec((128, 128), lambda i: (0, 0)),
            pl.BlockSpec((2, bm), lambda i: (0, i)),
        ],
        out_specs=(
            pl.BlockSpec((bm, 128), lambda i: (i, 0)),
            pl.BlockSpec((bm,), lambda i: (i,)),
        ),
    )(xpad, W1p, deg_parts)


def _conv1_post_body(lo_ref, hi_ref, y_ref, dinv_ref, b_ref, p_ref, pn_ref,
                     hs_ref, s_ref):
    bm = hs_ref.shape[0]
    i = pl.program_id(0)
    dinv = dinv_ref[...][:, None]
    b = b_ref[...]
    h_lo = jnp.maximum((lo_ref[0] + lo_ref[1] + y_ref[:, :64]) * dinv + b[:, :64], 0.0)
    h_hi = jnp.maximum((hi_ref[0] + hi_ref[1] + y_ref[:, 64:]) * dinv + b[:, 64:], 0.0)
    pr = p_ref[...] * pn_ref[0, 0]
    raw = (jnp.dot(h_lo, pr[:64, :], preferred_element_type=jnp.float32)
           + jnp.dot(h_hi, pr[64:, :], preferred_element_type=jnp.float32))
    row = i * bm + lax.broadcasted_iota(jnp.int32, (bm, 1), 0)
    s1 = jnp.where(row < N, jnp.tanh(raw), -2.0)
    hs_ref[:, :64] = h_lo * s1
    hs_ref[:, 64:] = h_hi * s1
    s_ref[...] = s1[:, 0]


def _conv1_post(acc_lo, acc_hi, y1, dinv1, b1p, p1c, p1norm_inv):
    bm = 2048
    return pl.pallas_call(
        _conv1_post_body,
        out_shape=(
            jax.ShapeDtypeStruct((NPAD, 128), jnp.float32),
            jax.ShapeDtypeStruct((NPAD,), jnp.float32),
        ),
        grid=(NPAD // bm,),
        in_specs=[
            pl.BlockSpec((2, bm, 64), lambda i: (0, i, 0)),
            pl.BlockSpec((2, bm, 64), lambda i: (0, i, 0)),
            pl.BlockSpec((bm, 128), lambda i: (i, 0)),
            pl.BlockSpec((bm,), lambda i: (i,)),
            pl.BlockSpec((1, 128), lambda i: (0, 0)),
            pl.BlockSpec((128, 1), lambda i: (0, 0)),
            pl.BlockSpec((1, 1), lambda i: (0, 0), memory_space=pltpu.SMEM),
        ],
        out_specs=(
            pl.BlockSpec((bm, 128), lambda i: (i, 0)),
            pl.BlockSpec((bm,), lambda i: (i,)),
        ),
    )(acc_lo, acc_hi, y1, dinv1, b1p, p1c, p1norm_inv)


def _cast_body(a_ref, o_ref):
    o_ref[...] = a_ref[...].astype(jnp.int32)


def _make_rank(n, b):
    def body(si_ref, sj_ref, o1_ref):
        i = pl.program_id(0)
        j = pl.program_id(1)

        @pl.when(j == 0)
        def _():
            o1_ref[...] = jnp.zeros_like(o1_ref)

        @pl.when(j > i)
        def _():
            sic = si_ref[0, :][:, None]
            sjr = sj_ref[0, :][None, :]
            o1_ref[...] += jnp.sum(jnp.where(sjr > sic, 1.0, 0.0), axis=1)

        @pl.when(j < i)
        def _():
            sic = si_ref[0, :][:, None]
            sjr = sj_ref[0, :][None, :]
            o1_ref[...] += jnp.sum(jnp.where(sjr >= sic, 1.0, 0.0), axis=1)

        @pl.when(j == i)
        def _():
            si = si_ref[0, :]
            sic = si[:, None]
            sjr = si[None, :]
            il = lax.broadcasted_iota(jnp.int32, (b, b), 0)
            jl = lax.broadcasted_iota(jnp.int32, (b, b), 1)
            contrib = (sjr > sic) | ((sjr == sic) & (jl < il))
            o1_ref[...] += jnp.sum(jnp.where(contrib, 1.0, 0.0), axis=1)

    def rank(s):
        s2d = s.reshape(1, n)
        o1 = pl.pallas_call(
            body,
            out_shape=jax.ShapeDtypeStruct((n,), jnp.float32),
            grid=(n // b, n // b),
            in_specs=[
                pl.BlockSpec((1, b), lambda i, j: (0, i)),
                pl.BlockSpec((1, b), lambda i, j: (0, j)),
            ],
            out_specs=pl.BlockSpec((b,), lambda i, j: (i,)),
        )(s2d, s2d)
        return pl.pallas_call(
            _cast_body,
            out_shape=jax.ShapeDtypeStruct((n,), jnp.int32),
        )(o1)

    return rank


_rank1 = _make_rank(NPAD, 1024)
_rank2 = _make_rank(NPOOL2, 1024)


def _conv2_pre_body(h1_ref, w_ref, deg_ref, z_ref, dinv_ref):
    bm = z_ref.shape[0]
    i = pl.program_id(0)
    deg = deg_ref[0, :] + deg_ref[1, :] + 1.0
    dinv = lax.rsqrt(deg)
    xw = jnp.dot(h1_ref[...], w_ref[...], preferred_element_type=jnp.float32)
    row = i * bm + lax.broadcasted_iota(jnp.int32, (bm, 1), 0)
    z_ref[...] = jnp.where(row < K1, xw * dinv[:, None], 0.0)
    dinv_ref[...] = dinv


def _conv2_pre(h1full, W2p, deg2_parts):
    bm = 1024
    return pl.pallas_call(
        _conv2_pre_body,
        out_shape=(
            jax.ShapeDtypeStruct((NPAD, 64), jnp.float32),
            jax.ShapeDtypeStruct((NPAD,), jnp.float32),
        ),
        grid=(NPAD // bm,),
        in_specs=[
            pl.BlockSpec((bm, 128), lambda i: (i, 0)),
            pl.BlockSpec((128, 64), lambda i: (0, 0)),
            pl.BlockSpec((2, bm), lambda i: (0, i)),
        ],
        out_specs=(
            pl.BlockSpec((bm, 64), lambda i: (i, 0)),
            pl.BlockSpec((bm,), lambda i: (i,)),
        ),
    )(h1full, W2p, deg2_parts)


def _conv2_post_body(acc_ref, z_ref, dinv_ref, b_ref, p_ref, pn_ref,
                     q_ref, s_ref):
    bm = q_ref.shape[0]
    i = pl.program_id(0)
    dinv = dinv_ref[...][:, None]
    h2 = jnp.maximum((acc_ref[0] + acc_ref[1] + z_ref[...]) * dinv + b_ref[...], 0.0)
    pr = p_ref[...] * pn_ref[0, 0]
    raw = jnp.dot(h2, pr, preferred_element_type=jnp.float32)
    row = i * bm + lax.broadcasted_iota(jnp.int32, (bm, 1), 0)
    s2 = jnp.where(row < K1, jnp.tanh(raw), -2.0)
    v = h2 * s2
    col = lax.broadcasted_iota(jnp.int32, (bm, 64), 1)
    cmask = col < H2_COLS
    m = jnp.max(jnp.where(cmask, v, -1e30), axis=1, keepdims=True)
    e = jnp.where(cmask, jnp.exp(v - m), 0.0)
    q_ref[...] = (v - m) - jnp.log(jnp.sum(e, axis=1, keepdims=True))
    s_ref[...] = s2[:, 0]


H2_COLS = 62


def _conv2_post(acc2, z2, dinv2, b2p, p2c, p2norm_inv):
    bm = 1024
    return pl.pallas_call(
        _conv2_post_body,
        out_shape=(
            jax.ShapeDtypeStruct((NPOOL2, 64), jnp.float32),
            jax.ShapeDtypeStruct((NPOOL2,), jnp.float32),
        ),
        grid=(NPOOL2 // bm,),
        in_specs=[
            pl.BlockSpec((2, bm, 64), lambda i: (0, i, 0)),
            pl.BlockSpec((bm, 64), lambda i: (i, 0)),
            pl.BlockSpec((bm,), lambda i: (i,)),
            pl.BlockSpec((1, 64), lambda i: (0, 0)),
            pl.BlockSpec((64, 1), lambda i: (0, 0)),
            pl.BlockSpec((1, 1), lambda i: (0, 0), memory_space=pltpu.SMEM),
        ],
        out_specs=(
            pl.BlockSpec((bm, 64), lambda i: (i, 0)),
            pl.BlockSpec((bm,), lambda i: (i,)),
        ),
    )(acc2, z2, dinv2, b2p, p2c, p2norm_inv)


def kernel(x, edge_index, W1, b1, p1, W2, b2, p2):
    f32 = jnp.float32
    src, dst = edge_index[0], edge_index[1]
    src_r = src.reshape(NW, WN, WL)
    dst_r = dst.reshape(NW, WN, WL)

    src_c1 = src_r
    dst_c1 = dst_r

    deg1_parts = _sc_hist(dst_r)
    xpad = jnp.pad(x, ((0, NPAD - N), (0, 0)))
    W1p = jnp.pad(W1, ((0, 0), (0, 128 - W1.shape[1])))
    y1, dinv1 = _conv1_pre(xpad, W1p, deg1_parts)
    acc_lo = _msgpass64_c1(y1[:, :64], src_c1, dst_c1)
    acc_hi = _msgpass64_c1(y1[:, 64:], src_c1, dst_c1)

    b1p = jnp.pad(b1, (0, 128 - b1.shape[0])).reshape(1, 128)
    p1c = jnp.pad(p1, (0, 128 - p1.shape[0])).reshape(128, 1)
    p1n = (1.0 / jnp.linalg.norm(p1)).reshape(1, 1)
    hs, s1 = _conv1_post(acc_lo, acc_hi, y1, dinv1, b1p, p1c, p1n)

    rank1 = _rank1(s1)
    h1full, gs_r, gd_r, deg2_parts = _sc_pool1(hs, rank1, src_r, dst_r)

    W2p = jnp.pad(W2, ((0, 2), (0, 64 - W2.shape[1])))
    z2, dinv2 = _conv2_pre(h1full, W2p, deg2_parts)
    acc2 = _msgpass64_c2(z2, gs_r, gd_r)

    b2p = jnp.pad(b2, (0, 64 - b2.shape[0])).reshape(1, 64)
    p2c = jnp.pad(p2, (0, 64 - p2.shape[0])).reshape(64, 1)
    p2n = (1.0 / jnp.linalg.norm(p2)).reshape(1, 1)
    q, s2 = _conv2_post(acc2[:, :NPOOL2], z2[:NPOOL2], dinv2[:NPOOL2],
                        b2p, p2c, p2n)

    rank2 = _rank2(s2)
    rankw2 = rank2.reshape(NW, 5, 32)
    out_full = _sc_scatter2(q, rankw2)
    return out_full[:K2, :H2_COLS]


# speedup vs baseline: 54.4783x; 1.0006x over previous
import functools

import jax
import jax.numpy as jnp
from jax import lax
from jax.experimental import pallas as pl
from jax.experimental.pallas import tpu as pltpu
from jax.experimental.pallas import tpu_sc as plsc

N = 10000
NPAD = 10240
E = 320000
NW = 32
WN = 125
WL = 80
K1 = 5000
K2 = 2500
NPOOL2 = 5120


def _mesh():
    return plsc.VectorSubcoreMesh(core_axis_name="c", subcore_axis_name="s")


_SC_PARAMS = pltpu.CompilerParams(use_tc_tiling_on_sc=False)
_SC_PARAMS_NL = pltpu.CompilerParams(use_tc_tiling_on_sc=False,
                                     needs_layout_passes=False)


@functools.partial(
    pl.kernel,
    out_type=jax.ShapeDtypeStruct((2, NPAD), jnp.float32),
    mesh=_mesh(),
    compiler_params=_SC_PARAMS,
    scratch_types=[
        pltpu.VMEM((WN, WL), jnp.int32),
        pltpu.VMEM((WL,), jnp.float32),
        pltpu.VMEM((NPAD // 16,), jnp.float32),
        pltpu.VMEM_SHARED((NPAD,), jnp.float32),
    ],
)
def _sc_hist(idx_hbm, out_hbm, idx_v, ones_v, z_v, acc_sh):
    c = lax.axis_index("c")
    s = lax.axis_index("s")
    wid = s * 2 + c
    sl = NPAD // 16

    def zbody(i, carry):
        z_v[pl.ds(i * 16, 16)] = jnp.zeros((16,), jnp.float32)
        return carry

    lax.fori_loop(0, sl // 16, zbody, 0)
    for u in range(WL // 16):
        ones_v[pl.ds(u * 16, 16)] = jnp.ones((16,), jnp.float32)
    pltpu.sync_copy(z_v, acc_sh.at[pl.ds(s * sl, sl)])
    plsc.subcore_barrier()

    pltpu.sync_copy(idx_hbm.at[wid], idx_v)

    def body(w, carry):
        pltpu.sync_copy(ones_v, acc_sh.at[idx_v.at[w]], add=True)
        return carry

    lax.fori_loop(0, WN, body, 0)
    plsc.subcore_barrier()
    pltpu.sync_copy(acc_sh.at[pl.ds(s * sl, sl)], out_hbm.at[c, pl.ds(s * sl, sl)])


def _make_msgpass(d_feat, wn, wl):
    @functools.partial(
        pl.kernel,
        out_type=jax.ShapeDtypeStruct((2, NPAD, d_feat), jnp.float32),
        mesh=_mesh(),
        compiler_params=_SC_PARAMS,
        scratch_types=[
            pltpu.VMEM((wn, wl), jnp.int32),
            pltpu.VMEM((wn, wl), jnp.int32),
            pltpu.VMEM((80, d_feat), jnp.float32),
            [pltpu.VMEM((wl, d_feat), jnp.float32) for _ in range(4)],
            pltpu.VMEM_SHARED((NPAD, d_feat), jnp.float32),
            [pltpu.SemaphoreType.DMA for _ in range(4)],
        ],
    )
    def msgpass(y_hbm, src_hbm, dst_hbm, out_hbm, sidx_v, didx_v, zb_v,
                rbufs, acc_sh, sems):
        c = lax.axis_index("c")
        s = lax.axis_index("s")
        wid = s * 2 + c
        rows = NPAD // 16

        def zbody(i, carry):
            for u in range(d_feat // 16):
                zb_v[i, pl.ds(u * 16, 16)] = jnp.zeros((16,), jnp.float32)
            return carry

        lax.fori_loop(0, 80, zbody, 0)
        for t in range(rows // 80):
            pltpu.sync_copy(zb_v, acc_sh.at[pl.ds(s * rows + t * 80, 80)])
        plsc.subcore_barrier()

        pltpu.sync_copy(src_hbm.at[wid], sidx_v)
        pltpu.sync_copy(dst_hbm.at[wid], didx_v)

        nbuf = 4
        dummy = y_hbm.at[pl.ds(0, wl)]
        for b in range(nbuf - 1):
            pltpu.async_copy(y_hbm.at[sidx_v.at[b]], rbufs[b], sems[b])

        def body(k, carry):
            w0 = nbuf * k
            for u in range(nbuf):
                w = w0 + u
                pltpu.make_async_copy(dummy, rbufs[u], sems[u]).wait()
                nxt = w + nbuf - 1
                bn = (u + nbuf - 1) % nbuf

                @pl.when(nxt < wn)
                def _():
                    pltpu.async_copy(y_hbm.at[sidx_v.at[nxt]], rbufs[bn],
                                     sems[bn])

                pltpu.sync_copy(rbufs[u], acc_sh.at[didx_v.at[w]], add=True)
            return carry

        lax.fori_loop(0, wn // nbuf, body, 0)
        for u in range(wn % nbuf):
            w = (wn // nbuf) * nbuf + u
            pltpu.make_async_copy(dummy, rbufs[w % nbuf], sems[w % nbuf]).wait()
            pltpu.sync_copy(rbufs[w % nbuf], acc_sh.at[didx_v.at[w]], add=True)

        plsc.subcore_barrier()
        pltpu.sync_copy(acc_sh.at[pl.ds(s * rows, rows)],
                        out_hbm.at[c, pl.ds(s * rows, rows)])

    return msgpass


_msgpass64_c1 = _make_msgpass(64, WN, WL)
_msgpass64_c2 = _msgpass64_c1


@functools.partial(
    pl.kernel,
    out_type=(
        jax.ShapeDtypeStruct((NPAD, 128), jnp.float32),
        jax.ShapeDtypeStruct((NW, WN, WL), jnp.int32),
        jax.ShapeDtypeStruct((NW, WN, WL), jnp.int32),
        jax.ShapeDtypeStruct((2, NPAD), jnp.float32),
    ),
    mesh=_mesh(),
    compiler_params=_SC_PARAMS_NL,
    scratch_types=[
        pltpu.VMEM((NPAD,), jnp.int32),
        pltpu.VMEM((64,), jnp.int32),
        pltpu.VMEM((64, 128), jnp.float32),
        pltpu.VMEM((WN, WL), jnp.int32),
        pltpu.VMEM((WN, WL), jnp.int32),
        pltpu.VMEM((WN, WL), jnp.int32),
        pltpu.VMEM((WN, WL), jnp.int32),
        pltpu.VMEM((WN, WL), jnp.float32),
        pltpu.VMEM((NPAD // 16,), jnp.float32),
        pltpu.VMEM_SHARED((NPAD,), jnp.float32),
        pltpu.SemaphoreType.DMA,
    ],
)
def _sc_pool1(hs_hbm, rank_hbm, src_hbm, dst_hbm,
              h1_hbm, gs_hbm, gd_hbm, deg2_hbm,
              rank_v, rkw_v, rowbuf_v, sidx_v, didx_v, gsv, gdv, ksv, z_v,
              deg_sh, sem):
    c = lax.axis_index("c")
    s = lax.axis_index("s")
    wid = s * 2 + c
    sl = NPAD // 16

    def zbody(i, carry):
        z_v[pl.ds(i * 16, 16)] = jnp.zeros((16,), jnp.float32)
        return carry

    lax.fori_loop(0, sl // 16, zbody, 0)
    pltpu.sync_copy(z_v, deg_sh.at[pl.ds(s * sl, sl)])
    plsc.subcore_barrier()

    pltpu.sync_copy(rank_hbm, rank_v)
    for w in range(5):
        for u in range(4):
            rkw_v[pl.ds(u * 16, 16)] = rank_v[pl.ds(wid * 320 + w * 64 + u * 16, 16)]
        pltpu.async_copy(hs_hbm.at[pl.ds(wid * 320 + w * 64, 64)], rowbuf_v,
                         sem).wait()
        pltpu.sync_copy(rowbuf_v, h1_hbm.at[rkw_v])

    pltpu.sync_copy(src_hbm.at[wid], sidx_v)
    pltpu.sync_copy(dst_hbm.at[wid], didx_v)

    def body(w, carry):
        for u in range(WL // 16):
            s16 = sidx_v[w, pl.ds(u * 16, 16)]
            d16 = didx_v[w, pl.ds(u * 16, 16)]
            gs16 = plsc.load_gather(rank_v, [s16])
            gd16 = plsc.load_gather(rank_v, [d16])
            ks16 = jnp.where(gs16 < K1, 1.0, 0.0).astype(jnp.float32)
            gsv[w, pl.ds(u * 16, 16)] = gs16
            gdv[w, pl.ds(u * 16, 16)] = gd16
            ksv[w, pl.ds(u * 16, 16)] = ks16
        pltpu.sync_copy(ksv.at[w], deg_sh.at[gdv.at[w]], add=True)
        return carry

    lax.fori_loop(0, WN, body, 0)
    pltpu.sync_copy(gsv, gs_hbm.at[wid])
    pltpu.sync_copy(gdv, gd_hbm.at[wid])
    plsc.subcore_barrier()
    pltpu.sync_copy(deg_sh.at[pl.ds(s * sl, sl)], deg2_hbm.at[c, pl.ds(s * sl, sl)])


@functools.partial(
    pl.kernel,
    out_type=jax.ShapeDtypeStruct((NPOOL2, 64), jnp.float32),
    mesh=_mesh(),
    compiler_params=_SC_PARAMS,
    scratch_types=[
        pltpu.VMEM((5, 32), jnp.int32),
        pltpu.VMEM((32, 64), jnp.float32),
        pltpu.SemaphoreType.DMA,
    ],
)
def _sc_scatter2(q_hbm, rankw_hbm, out_hbm, rkw_v, rowbuf_v, sem):
    c = lax.axis_index("c")
    s = lax.axis_index("s")
    wid = s * 2 + c
    pltpu.sync_copy(rankw_hbm.at[wid], rkw_v)
    for w in range(5):
        pltpu.async_copy(q_hbm.at[pl.ds(wid * 160 + w * 32, 32)], rowbuf_v,
                         sem).wait()
        pltpu.sync_copy(rowbuf_v, out_hbm.at[rkw_v.at[w]])


def _conv1_pre_body(x_ref, w_ref, deg_ref, y_ref, dinv_ref):
    deg = deg_ref[0, :] + deg_ref[1, :] + 1.0
    dinv = lax.rsqrt(deg)
    xw = jnp.dot(x_ref[...], w_ref[...], preferred_element_type=jnp.float32)
    y_ref[...] = xw * dinv[:, None]
    dinv_ref[...] = dinv


def _conv1_pre(xpad, W1p, deg_parts):
    bm = 2048
    return pl.pallas_call(
        _conv1_pre_body,
        out_shape=(
            jax.ShapeDtypeStruct((NPAD, 128), jnp.float32),
            jax.ShapeDtypeStruct((NPAD,), jnp.float32),
        ),
        grid=(NPAD // bm,),
        in_specs=[
            pl.BlockSpec((bm, 128), lambda i: (i, 0)),
            pl.BlockSpec((128, 128), lambda i: (0, 0)),
            pl.BlockSpec((2, bm), lambda i: (0, i)),
        ],
        out_specs=(
            pl.BlockSpec((bm, 128), lambda i: (i, 0)),
            pl.BlockSpec((bm,), lambda i: (i,)),
        ),
    )(xpad, W1p, deg_parts)


def _conv1_post_body(lo_ref, hi_ref, y_ref, dinv_ref, b_ref, p_ref, pn_ref,
                     hs_ref, s_ref):
    bm = hs_ref.shape[0]
    i = pl.program_id(0)
    dinv = dinv_ref[...][:, None]
    b = b_ref[...]
    h_lo = jnp.maximum((lo_ref[0] + lo_ref[1] + y_ref[:, :64]) * dinv + b[:, :64], 0.0)
    h_hi = jnp.maximum((hi_ref[0] + hi_ref[1] + y_ref[:, 64:]) * dinv + b[:, 64:], 0.0)
    pr = p_ref[...] * pn_ref[0, 0]
    raw = (jnp.dot(h_lo, pr[:64, :], preferred_element_type=jnp.float32)
           + jnp.dot(h_hi, pr[64:, :], preferred_element_type=jnp.float32))
    row = i * bm + lax.broadcasted_iota(jnp.int32, (bm, 1), 0)
    s1 = jnp.where(row < N, jnp.tanh(raw), -2.0)
    hs_ref[:, :64] = h_lo * s1
    hs_ref[:, 64:] = h_hi * s1
    s_ref[...] = s1[:, 0]


def _conv1_post(acc_lo, acc_hi, y1, dinv1, b1p, p1c, p1norm_inv):
    bm = 2048
    return pl.pallas_call(
        _conv1_post_body,
        out_shape=(
            jax.ShapeDtypeStruct((NPAD, 128), jnp.float32),
            jax.ShapeDtypeStruct((NPAD,), jnp.float32),
        ),
        grid=(NPAD // bm,),
        in_specs=[
            pl.BlockSpec((2, bm, 64), lambda i: (0, i, 0)),
            pl.BlockSpec((2, bm, 64), lambda i: (0, i, 0)),
            pl.BlockSpec((bm, 128), lambda i: (i, 0)),
            pl.BlockSpec((bm,), lambda i: (i,)),
            pl.BlockSpec((1, 128), lambda i: (0, 0)),
            pl.BlockSpec((128, 1), lambda i: (0, 0)),
            pl.BlockSpec((1, 1), lambda i: (0, 0), memory_space=pltpu.SMEM),
        ],
        out_specs=(
            pl.BlockSpec((bm, 128), lambda i: (i, 0)),
            pl.BlockSpec((bm,), lambda i: (i,)),
        ),
    )(acc_lo, acc_hi, y1, dinv1, b1p, p1c, p1norm_inv)


def _cast_body(a_ref, o_ref):
    o_ref[...] = a_ref[...].astype(jnp.int32)


def _make_rank(n, b):
    def body(si_ref, sj_ref, o1_ref):
        i = pl.program_id(0)
        j = pl.program_id(1)

        @pl.when(j == 0)
        def _():
            o1_ref[...] = jnp.zeros_like(o1_ref)

        @pl.when(j > i)
        def _():
            sic = si_ref[0, :][:, None]
            sjr = sj_ref[0, :][None, :]
            o1_ref[...] += jnp.sum(jnp.where(sjr > sic, 1.0, 0.0), axis=1)

        @pl.when(j < i)
        def _():
            sic = si_ref[0, :][:, None]
            sjr = sj_ref[0, :][None, :]
            o1_ref[...] += jnp.sum(jnp.where(sjr >= sic, 1.0, 0.0), axis=1)

        @pl.when(j == i)
        def _():
            si = si_ref[0, :]
            sic = si[:, None]
            sjr = si[None, :]
            il = lax.broadcasted_iota(jnp.int32, (b, b), 0)
            jl = lax.broadcasted_iota(jnp.int32, (b, b), 1)
            contrib = (sjr > sic) | ((sjr == sic) & (jl < il))
            o1_ref[...] += jnp.sum(jnp.where(contrib, 1.0, 0.0), axis=1)

    def rank(s):
        s2d = s.reshape(1, n)
        o1 = pl.pallas_call(
            body,
            out_shape=jax.ShapeDtypeStruct((n,), jnp.float32),
            grid=(n // b, n // b),
            in_specs=[
                pl.BlockSpec((1, b), lambda i, j: (0, i)),
                pl.BlockSpec((1, b), lambda i, j: (0, j)),
            ],
            out_specs=pl.BlockSpec((b,), lambda i, j: (i,)),
        )(s2d, s2d)
        return pl.pallas_call(
            _cast_body,
            out_shape=jax.ShapeDtypeStruct((n,), jnp.int32),
        )(o1)

    return rank


_rank1 = _make_rank(NPAD, 1024)
_rank2 = _make_rank(NPOOL2, 1024)


def _conv2_pre_body(h1_ref, w_ref, deg_ref, z_ref, dinv_ref):
    bm = z_ref.shape[0]
    i = pl.program_id(0)
    deg = deg_ref[0, :] + deg_ref[1, :] + 1.0
    dinv = lax.rsqrt(deg)
    xw = jnp.dot(h1_ref[...], w_ref[...], preferred_element_type=jnp.float32)
    row = i * bm + lax.broadcasted_iota(jnp.int32, (bm, 1), 0)
    z_ref[...] = jnp.where(row < K1, xw * dinv[:, None], 0.0)
    dinv_ref[...] = dinv


def _conv2_pre(h1full, W2p, deg2_parts):
    bm = 1024
    return pl.pallas_call(
        _conv2_pre_body,
        out_shape=(
            jax.ShapeDtypeStruct((NPAD, 64), jnp.float32),
            jax.ShapeDtypeStruct((NPAD,), jnp.float32),
        ),
        grid=(NPAD // bm,),
        in_specs=[
            pl.BlockSpec((bm, 128), lambda i: (i, 0)),
            pl.BlockSpec((128, 64), lambda i: (0, 0)),
            pl.BlockSpec((2, bm), lambda i: (0, i)),
        ],
        out_specs=(
            pl.BlockSpec((bm, 64), lambda i: (i, 0)),
            pl.BlockSpec((bm,), lambda i: (i,)),
        ),
    )(h1full, W2p, deg2_parts)


def _conv2_post_body(acc_ref, z_ref, dinv_ref, b_ref, p_ref, pn_ref,
                     q_ref, s_ref):
    bm = q_ref.shape[0]
    i = pl.program_id(0)
    dinv = dinv_ref[...][:, None]
    h2 = jnp.maximum((acc_ref[0] + acc_ref[1] + z_ref[...]) * dinv + b_ref[...], 0.0)
    pr = p_ref[...] * pn_ref[0, 0]
    raw = jnp.dot(h2, pr, preferred_element_type=jnp.float32)
    row = i * bm + lax.broadcasted_iota(jnp.int32, (bm, 1), 0)
    s2 = jnp.where(row < K1, jnp.tanh(raw), -2.0)
    v = h2 * s2
    col = lax.broadcasted_iota(jnp.int32, (bm, 64), 1)
    cmask = col < H2_COLS
    m = jnp.max(jnp.where(cmask, v, -1e30), axis=1, keepdims=True)
    e = jnp.where(cmask, jnp.exp(v - m), 0.0)
    q_ref[...] = (v - m) - jnp.log(jnp.sum(e, axis=1, keepdims=True))
    s_ref[...] = s2[:, 0]


H2_COLS = 62


def _conv2_post(acc2, z2, dinv2, b2p, p2c, p2norm_inv):
    bm = 1024
    return pl.pallas_call(
        _conv2_post_body,
        out_shape=(
            jax.ShapeDtypeStruct((NPOOL2, 64), jnp.float32),
            jax.ShapeDtypeStruct((NPOOL2,), jnp.float32),
        ),
        grid=(NPOOL2 // bm,),
        in_specs=[
            pl.BlockSpec((2, bm, 64), lambda i: (0, i, 0)),
            pl.BlockSpec((bm, 64), lambda i: (i, 0)),
            pl.BlockSpec((bm,), lambda i: (i,)),
            pl.BlockSpec((1, 64), lambda i: (0, 0)),
            pl.BlockSpec((64, 1), lambda i: (0, 0)),
            pl.BlockSpec((1, 1), lambda i: (0, 0), memory_space=pltpu.SMEM),
        ],
        out_specs=(
            pl.BlockSpec((bm, 64), lambda i: (i, 0)),
            pl.BlockSpec((bm,), lambda i: (i,)),
        ),
    )(acc2, z2, dinv2, b2p, p2c, p2norm_inv)


def kernel(x, edge_index, W1, b1, p1, W2, b2, p2):
    f32 = jnp.float32
    src, dst = edge_index[0], edge_index[1]
    src_r = src.reshape(NW, WN, WL)
    dst_r = dst.reshape(NW, WN, WL)

    src_c1 = src_r
    dst_c1 = dst_r

    deg1_parts = _sc_hist(dst_r)
    xpad = jnp.pad(x, ((0, NPAD - N), (0, 0)))
    W1p = jnp.pad(W1, ((0, 0), (0, 128 - W1.shape[1])))
    y1, dinv1 = _conv1_pre(xpad, W1p, deg1_parts)
    acc_lo = _msgpass64_c1(y1[:, :64], src_c1, dst_c1)
    acc_hi = _msgpass64_c1(y1[:, 64:], src_c1, dst_c1)

    b1p = jnp.pad(b1, (0, 128 - b1.shape[0])).reshape(1, 128)
    p1c = jnp.pad(p1, (0, 128 - p1.shape[0])).reshape(128, 1)
    p1n = (1.0 / jnp.linalg.norm(p1)).reshape(1, 1)
    hs, s1 = _conv1_post(acc_lo, acc_hi, y1, dinv1, b1p, p1c, p1n)

    rank1 = _rank1(s1)
    h1full, gs_r, gd_r, deg2_parts = _sc_pool1(hs, rank1, src_r, dst_r)

    W2p = jnp.pad(W2, ((0, 2), (0, 64 - W2.shape[1])))
    z2, dinv2 = _conv2_pre(h1full, W2p, deg2_parts)
    acc2 = _msgpass64_c2(z2, gs_r, gd_r)

    b2p = jnp.pad(b2, (0, 64 - b2.shape[0])).reshape(1, 64)
    p2c = jnp.pad(p2, (0, 64 - p2.shape[0])).reshape(64, 1)
    p2n = (1.0 / jnp.linalg.norm(p2)).reshape(1, 1)
    q, s2 = _conv2_post(acc2[:, :NPOOL2], z2[:NPOOL2], dinv2[:NPOOL2],
                        b2p, p2c, p2n)

    rank2 = _rank2(s2)
    rankw2 = rank2.reshape(NW, 5, 32)
    out_full = _sc_scatter2(q, rankw2)
    return out_full[:K2, :H2_COLS]
